# Initial kernel scaffold; baseline (speedup 1.0000x reference)
import jax
import jax.numpy as jnp
from jax.experimental import pallas as pl


def kernel(feature, btn_w, btn_b, g1_theta_w, g1_theta_b, g1_phi_w, g1_phi_b, g1_mlp_w1, g1_mlp_b1, g1_mlp_w2, g1_mlp_b2, g2_theta_w, g2_theta_b, g2_phi_w, g2_phi_b, g2_mlp_w1, g2_mlp_b1, g2_mlp_w2, g2_mlp_b2, lin_w, lin_b):
    raise NotImplementedError("write your pallas kernel here")


# speedup vs baseline: 6.4707x; 6.4707x over previous
import functools
import jax
import jax.numpy as jnp
from jax import lax
from jax.experimental import pallas as pl
from jax.experimental.pallas import tpu as pltpu

N = 10000
IN_FEATS = 256
D = 64
HID = 64
K = 40

R_PREP = 1000
R_KNN = 200
R_EDGE = 200


def _leaky(x):
    return jnp.where(x >= 0, x, 0.01 * x)


def _prep_body(feat, btn_w, btn_b, tpw1, tpb1, tw1, tpw2, tpb2, tw2,
               f_o, sq_o, p1_o, q1_o, p2_o, q2_o):
    f = jnp.dot(feat[...], btn_w[...], preferred_element_type=jnp.float32) + btn_b[...]
    f_o[...] = f
    sq_o[...] = jnp.sum(f * f, axis=1, keepdims=True)
    p1_o[...] = jnp.dot(f, tpw1[...], preferred_element_type=jnp.float32) + tpb1[...]
    q1_o[...] = jnp.dot(f, tw1[...], preferred_element_type=jnp.float32)
    p2_o[...] = jnp.dot(f, tpw2[...], preferred_element_type=jnp.float32) + tpb2[...]
    q2_o[...] = jnp.dot(f, tw2[...], preferred_element_type=jnp.float32)


def _prep(feature, btn_w, btn_b, tpw1, tpb1, tw1, tpw2, tpb2, tw2):
    nb = N // R_PREP
    full = lambda shape: pl.BlockSpec(shape, lambda i: tuple(0 for _ in shape))
    out_shapes = (
        jax.ShapeDtypeStruct((N, D), jnp.float32),
        jax.ShapeDtypeStruct((N, 1), jnp.float32),
        jax.ShapeDtypeStruct((N, D), jnp.float32),
        jax.ShapeDtypeStruct((N, D), jnp.float32),
        jax.ShapeDtypeStruct((N, D), jnp.float32),
        jax.ShapeDtypeStruct((N, D), jnp.float32),
    )
    blk_nd = pl.BlockSpec((R_PREP, D), lambda i: (i, 0))
    return pl.pallas_call(
        _prep_body,
        grid=(nb,),
        in_specs=[
            pl.BlockSpec((R_PREP, IN_FEATS), lambda i: (i, 0)),
            full((IN_FEATS, D)), full((1, D)),
            full((D, D)), full((1, D)), full((D, D)),
            full((D, D)), full((1, D)), full((D, D)),
        ],
        out_specs=(
            blk_nd, pl.BlockSpec((R_PREP, 1), lambda i: (i, 0)),
            blk_nd, blk_nd, blk_nd, blk_nd,
        ),
        out_shape=out_shapes,
    )(feature, btn_w, btn_b, tpw1, tpb1, tw1, tpw2, tpb2, tw2)


def _knn_body(f_blk, sq_blk, f_full, sqT, idx_o):
    g = lax.dot_general(f_blk[...], f_full[...],
                        (((1,), (1,)), ((), ())),
                        preferred_element_type=jnp.float32)
    dist = (sq_blk[...] + sqT[...]) - 2.0 * g
    iota = lax.broadcasted_iota(jnp.int32, (R_KNN, N), 1)
    big = jnp.int32(2 ** 30)
    inf = jnp.float32(jnp.inf)
    for t in range(K):
        m = jnp.min(dist, axis=1, keepdims=True)
        amin = jnp.min(jnp.where(dist == m, iota, big), axis=1, keepdims=True)
        idx_o[:, t:t + 1] = amin
        dist = jnp.where(iota == amin, inf, dist)


def _knn(f, sq, sqT):
    nb = N // R_KNN
    return pl.pallas_call(
        _knn_body,
        grid=(nb,),
        in_specs=[
            pl.BlockSpec((R_KNN, D), lambda i: (i, 0)),
            pl.BlockSpec((R_KNN, 1), lambda i: (i, 0)),
            pl.BlockSpec((N, D), lambda i: (0, 0)),
            pl.BlockSpec((1, N), lambda i: (0, 0)),
        ],
        out_specs=pl.BlockSpec((R_KNN, K), lambda i: (i, 0)),
        out_shape=jax.ShapeDtypeStruct((N, K), jnp.int32),
    )(f, sq, f, sqT)


def _edge_body(q1g, q2g, f9, p1, p2, feat,
               w11, b11, w12, b12, w21, b21, w22, b22,
               lw1, lw2, lw3, lin_b, out_o):
    er20 = lax.broadcasted_iota(jnp.int32, (R_EDGE * 20, R_EDGE), 0) // 20
    ec20 = lax.broadcasted_iota(jnp.int32, (R_EDGE * 20, R_EDGE), 1)
    e20 = (er20 == ec20).astype(jnp.float32)

    def conv(qg, p, w1, b1, w2, b2):
        pexp = jnp.dot(e20, p[...], preferred_element_type=jnp.float32)
        h = _leaky(pexp - qg[...])
        h = _leaky(jnp.dot(h, w1[...], preferred_element_type=jnp.float32) + b1[...])
        h = jnp.dot(h, w2[...], preferred_element_type=jnp.float32) + b2[...]
        return jnp.max(h.reshape(R_EDGE, 20, D), axis=1)

    h1 = conv(q1g, p1, w11, b11, w12, b12)
    h2 = conv(q2g, p2, w21, b21, w22, b22)
    hm = jnp.max(f9[...].reshape(R_EDGE, 9, D), axis=1)
    out = (jnp.dot(h1, lw1[...], preferred_element_type=jnp.float32)
           + jnp.dot(h2, lw2[...], preferred_element_type=jnp.float32)
           + jnp.dot(hm, lw3[...], preferred_element_type=jnp.float32)
           + lin_b[...] + feat[...])
    out_o[...] = out


def _edge(q1g, q2g, f9, p1, p2, feat,
          w11, b11, w12, b12, w21, b21, w22, b22, lw1, lw2, lw3, lin_b):
    nb = N // R_EDGE
    full = lambda shape: pl.BlockSpec(shape, lambda i: tuple(0 for _ in shape))
    return pl.pallas_call(
        _edge_body,
        grid=(nb,),
        in_specs=[
            pl.BlockSpec((R_EDGE * 20, D), lambda i: (i, 0)),
            pl.BlockSpec((R_EDGE * 20, D), lambda i: (i, 0)),
            pl.BlockSpec((R_EDGE * 9, D), lambda i: (i, 0)),
            pl.BlockSpec((R_EDGE, D), lambda i: (i, 0)),
            pl.BlockSpec((R_EDGE, D), lambda i: (i, 0)),
            pl.BlockSpec((R_EDGE, IN_FEATS), lambda i: (i, 0)),
            full((D, HID)), full((1, HID)), full((HID, D)), full((1, D)),
            full((D, HID)), full((1, HID)), full((HID, D)), full((1, D)),
            full((D, IN_FEATS)), full((D, IN_FEATS)), full((D, IN_FEATS)),
            full((1, IN_FEATS)),
        ],
        out_specs=pl.BlockSpec((R_EDGE, IN_FEATS), lambda i: (i, 0)),
        out_shape=jax.ShapeDtypeStruct((N, IN_FEATS), jnp.float32),
    )(q1g, q2g, f9, p1, p2, feat,
      w11, b11, w12, b12, w21, b21, w22, b22, lw1, lw2, lw3, lin_b)


def kernel(feature, btn_w, btn_b,
           g1_theta_w, g1_theta_b, g1_phi_w, g1_phi_b,
           g1_mlp_w1, g1_mlp_b1, g1_mlp_w2, g1_mlp_b2,
           g2_theta_w, g2_theta_b, g2_phi_w, g2_phi_b,
           g2_mlp_w1, g2_mlp_b1, g2_mlp_w2, g2_mlp_b2,
           lin_w, lin_b):
    r = lambda b: b.reshape(1, -1)
    tpw1 = g1_theta_w + g1_phi_w
    tpb1 = r(g1_theta_b + g1_phi_b)
    tpw2 = g2_theta_w + g2_phi_w
    tpb2 = r(g2_theta_b + g2_phi_b)

    f, sq, p1, q1, p2, q2 = _prep(feature, btn_w, r(btn_b),
                                  tpw1, tpb1, g1_theta_w,
                                  tpw2, tpb2, g2_theta_w)
    sqT = sq.reshape(1, N)
    idx = _knn(f, sq, sqT)

    idx20 = idx[:, :20].reshape(-1)
    idx40d = idx[:, ::2].reshape(-1)
    idx9 = idx[:, :9].reshape(-1)
    q1g = jnp.take(q1, idx20, axis=0)
    q2g = jnp.take(q2, idx40d, axis=0)
    f9 = jnp.take(f, idx9, axis=0)

    out = _edge(q1g, q2g, f9, p1, p2, feature,
                g1_mlp_w1, r(g1_mlp_b1), g1_mlp_w2, r(g1_mlp_b2),
                g2_mlp_w1, r(g2_mlp_b1), g2_mlp_w2, r(g2_mlp_b2),
                lin_w[0:D, :], lin_w[D:2 * D, :], lin_w[2 * D:3 * D, :],
                r(lin_b))
    return out


# speedup vs baseline: 10.7418x; 1.6601x over previous
import functools
import jax
import jax.numpy as jnp
from jax import lax
from jax.experimental import pallas as pl
from jax.experimental.pallas import tpu as pltpu

N = 10000
IN_FEATS = 256
D = 64
HID = 64
K = 40

R_PREP = 1000
R_KNN = 200
R_EDGE = 200


def _leaky(x):
    return jnp.where(x >= 0, x, 0.01 * x)


def _prep_body(feat, btn_w, btn_b, tpw1, tpb1, tw1, tpw2, tpb2, tw2,
               f_o, sq_o, p1_o, q1_o, p2_o, q2_o):
    f = jnp.dot(feat[...], btn_w[...], preferred_element_type=jnp.float32) + btn_b[...]
    f_o[...] = f
    sq_o[...] = jnp.sum(f * f, axis=1, keepdims=True)
    p1_o[...] = jnp.dot(f, tpw1[...], preferred_element_type=jnp.float32) + tpb1[...]
    q1_o[...] = jnp.dot(f, tw1[...], preferred_element_type=jnp.float32)
    p2_o[...] = jnp.dot(f, tpw2[...], preferred_element_type=jnp.float32) + tpb2[...]
    q2_o[...] = jnp.dot(f, tw2[...], preferred_element_type=jnp.float32)


def _prep(feature, btn_w, btn_b, tpw1, tpb1, tw1, tpw2, tpb2, tw2):
    nb = N // R_PREP
    full = lambda shape: pl.BlockSpec(shape, lambda i: tuple(0 for _ in shape))
    out_shapes = (
        jax.ShapeDtypeStruct((N, D), jnp.float32),
        jax.ShapeDtypeStruct((N, 1), jnp.float32),
        jax.ShapeDtypeStruct((N, D), jnp.float32),
        jax.ShapeDtypeStruct((N, D), jnp.float32),
        jax.ShapeDtypeStruct((N, D), jnp.float32),
        jax.ShapeDtypeStruct((N, D), jnp.float32),
    )
    blk_nd = pl.BlockSpec((R_PREP, D), lambda i: (i, 0))
    return pl.pallas_call(
        _prep_body,
        grid=(nb,),
        in_specs=[
            pl.BlockSpec((R_PREP, IN_FEATS), lambda i: (i, 0)),
            full((IN_FEATS, D)), full((1, D)),
            full((D, D)), full((1, D)), full((D, D)),
            full((D, D)), full((1, D)), full((D, D)),
        ],
        out_specs=(
            blk_nd, pl.BlockSpec((R_PREP, 1), lambda i: (i, 0)),
            blk_nd, blk_nd, blk_nd, blk_nd,
        ),
        out_shape=out_shapes,
    )(feature, btn_w, btn_b, tpw1, tpb1, tw1, tpw2, tpb2, tw2)


NLANE = 128
NCHUNK = (N + NLANE - 1) // NLANE
B_INS = 6


def _knn_body(f_blk, sq_blk, f_full, sqT, idx_o, dscr):
    R = R_KNN
    g = lax.dot_general(f_blk[...], f_full[...],
                        (((1,), (1,)), ((), ())),
                        preferred_element_type=jnp.float32)
    dist = (sq_blk[...] + sqT[...]) - 2.0 * g
    big = jnp.int32(2 ** 30)
    inf = jnp.float32(jnp.inf)

    pad = NCHUNK * NLANE - N
    dscr[...] = jnp.concatenate(
        [dist, jnp.full((R, pad), inf, dtype=jnp.float32)], axis=1)

    lane = lax.broadcasted_iota(jnp.int32, (R, NLANE), 1)

    def lex_lt(av, ai, bv, bi):
        return (av < bv) | ((av == bv) & (ai < bi))

    def ins_step(c, st):
        x = dscr[:, pl.ds(c * NLANE, NLANE)]
        xi = lane + c * NLANE
        cv, ci = x, xi
        out = []
        for b in range(B_INS):
            vb, ib = st[2 * b], st[2 * b + 1]
            lt = lex_lt(cv, ci, vb, ib)
            nv = jnp.where(lt, cv, vb)
            ni = jnp.where(lt, ci, ib)
            cv, ci = jnp.where(lt, vb, cv), jnp.where(lt, ib, ci)
            out += [nv, ni]
        return tuple(out)

    init = []
    for b in range(B_INS):
        init += [jnp.full((R, NLANE), inf, dtype=jnp.float32),
                 jnp.full((R, NLANE), big, dtype=jnp.int32)]
    st = lax.fori_loop(0, NCHUNK, ins_step, tuple(init), unroll=2)

    cand_v = jnp.concatenate([st[2 * b] for b in range(B_INS)], axis=1)
    cand_i = jnp.concatenate([st[2 * b + 1] for b in range(B_INS)], axis=1)

    cols = []
    work = cand_v
    m = None
    amin = None
    for t in range(K):
        m = jnp.min(work, axis=1, keepdims=True)
        amin = jnp.min(jnp.where(work == m, cand_i, big), axis=1, keepdims=True)
        cols.append(amin)
        work = jnp.where(cand_i == amin, inf, work)
    idx_fast = jnp.concatenate(cols, axis=1)

    ovf = jnp.any(lex_lt(st[2 * (B_INS - 1)], st[2 * (B_INS - 1) + 1], m, amin))

    def slow():
        iota = lax.broadcasted_iota(jnp.int32, (R, N), 1)
        d = dist
        cs = []
        for t in range(K):
            mm = jnp.min(d, axis=1, keepdims=True)
            am = jnp.min(jnp.where(d == mm, iota, big), axis=1, keepdims=True)
            cs.append(am)
            d = jnp.where(iota == am, inf, d)
        return jnp.concatenate(cs, axis=1)

    idx_o[...] = lax.cond(ovf, slow, lambda: idx_fast)


def _knn(f, sq, sqT):
    nb = N // R_KNN
    return pl.pallas_call(
        _knn_body,
        grid=(nb,),
        in_specs=[
            pl.BlockSpec((R_KNN, D), lambda i: (i, 0)),
            pl.BlockSpec((R_KNN, 1), lambda i: (i, 0)),
            pl.BlockSpec((N, D), lambda i: (0, 0)),
            pl.BlockSpec((1, N), lambda i: (0, 0)),
        ],
        out_specs=pl.BlockSpec((R_KNN, K), lambda i: (i, 0)),
        out_shape=jax.ShapeDtypeStruct((N, K), jnp.int32),
        scratch_shapes=[pltpu.VMEM((R_KNN, NCHUNK * NLANE), jnp.float32)],
    )(f, sq, f, sqT)


def _edge_body(q1g, q2g, f9, p1, p2, feat,
               w11, b11, w12, b12, w21, b21, w22, b22,
               lw1, lw2, lw3, lin_b, out_o):
    er20 = lax.broadcasted_iota(jnp.int32, (R_EDGE * 20, R_EDGE), 0) // 20
    ec20 = lax.broadcasted_iota(jnp.int32, (R_EDGE * 20, R_EDGE), 1)
    e20 = (er20 == ec20).astype(jnp.float32)

    def conv(qg, p, w1, b1, w2, b2):
        pexp = jnp.dot(e20, p[...], preferred_element_type=jnp.float32)
        h = _leaky(pexp - qg[...])
        h = _leaky(jnp.dot(h, w1[...], preferred_element_type=jnp.float32) + b1[...])
        h = jnp.dot(h, w2[...], preferred_element_type=jnp.float32) + b2[...]
        return jnp.max(h.reshape(R_EDGE, 20, D), axis=1)

    h1 = conv(q1g, p1, w11, b11, w12, b12)
    h2 = conv(q2g, p2, w21, b21, w22, b22)
    hm = jnp.max(f9[...].reshape(R_EDGE, 9, D), axis=1)
    out = (jnp.dot(h1, lw1[...], preferred_element_type=jnp.float32)
           + jnp.dot(h2, lw2[...], preferred_element_type=jnp.float32)
           + jnp.dot(hm, lw3[...], preferred_element_type=jnp.float32)
           + lin_b[...] + feat[...])
    out_o[...] = out


def _edge(q1g, q2g, f9, p1, p2, feat,
          w11, b11, w12, b12, w21, b21, w22, b22, lw1, lw2, lw3, lin_b):
    nb = N // R_EDGE
    full = lambda shape: pl.BlockSpec(shape, lambda i: tuple(0 for _ in shape))
    return pl.pallas_call(
        _edge_body,
        grid=(nb,),
        in_specs=[
            pl.BlockSpec((R_EDGE * 20, D), lambda i: (i, 0)),
            pl.BlockSpec((R_EDGE * 20, D), lambda i: (i, 0)),
            pl.BlockSpec((R_EDGE * 9, D), lambda i: (i, 0)),
            pl.BlockSpec((R_EDGE, D), lambda i: (i, 0)),
            pl.BlockSpec((R_EDGE, D), lambda i: (i, 0)),
            pl.BlockSpec((R_EDGE, IN_FEATS), lambda i: (i, 0)),
            full((D, HID)), full((1, HID)), full((HID, D)), full((1, D)),
            full((D, HID)), full((1, HID)), full((HID, D)), full((1, D)),
            full((D, IN_FEATS)), full((D, IN_FEATS)), full((D, IN_FEATS)),
            full((1, IN_FEATS)),
        ],
        out_specs=pl.BlockSpec((R_EDGE, IN_FEATS), lambda i: (i, 0)),
        out_shape=jax.ShapeDtypeStruct((N, IN_FEATS), jnp.float32),
    )(q1g, q2g, f9, p1, p2, feat,
      w11, b11, w12, b12, w21, b21, w22, b22, lw1, lw2, lw3, lin_b)


def kernel(feature, btn_w, btn_b,
           g1_theta_w, g1_theta_b, g1_phi_w, g1_phi_b,
           g1_mlp_w1, g1_mlp_b1, g1_mlp_w2, g1_mlp_b2,
           g2_theta_w, g2_theta_b, g2_phi_w, g2_phi_b,
           g2_mlp_w1, g2_mlp_b1, g2_mlp_w2, g2_mlp_b2,
           lin_w, lin_b):
    r = lambda b: b.reshape(1, -1)
    tpw1 = g1_theta_w + g1_phi_w
    tpb1 = r(g1_theta_b + g1_phi_b)
    tpw2 = g2_theta_w + g2_phi_w
    tpb2 = r(g2_theta_b + g2_phi_b)

    f, sq, p1, q1, p2, q2 = _prep(feature, btn_w, r(btn_b),
                                  tpw1, tpb1, g1_theta_w,
                                  tpw2, tpb2, g2_theta_w)
    sqT = sq.reshape(1, N)
    idx = _knn(f, sq, sqT)

    idx20 = idx[:, :20].reshape(-1)
    idx40d = idx[:, ::2].reshape(-1)
    idx9 = idx[:, :9].reshape(-1)
    q1g = jnp.take(q1, idx20, axis=0)
    q2g = jnp.take(q2, idx40d, axis=0)
    f9 = jnp.take(f, idx9, axis=0)

    out = _edge(q1g, q2g, f9, p1, p2, feature,
                g1_mlp_w1, r(g1_mlp_b1), g1_mlp_w2, r(g1_mlp_b2),
                g2_mlp_w1, r(g2_mlp_b1), g2_mlp_w2, r(g2_mlp_b2),
                lin_w[0:D, :], lin_w[D:2 * D, :], lin_w[2 * D:3 * D, :],
                r(lin_b))
    return out


# speedup vs baseline: 13.8218x; 1.2867x over previous
import functools
import jax
import jax.numpy as jnp
from jax import lax
from jax.experimental import pallas as pl
from jax.experimental.pallas import tpu as pltpu
from jax.experimental.pallas import tpu_sc as plsc

N = 10000
IN_FEATS = 256
D = 64
HID = 64
K = 40

R_PREP = 1000
R_KNN = 200
R_EDGE = 200


def _leaky(x):
    return jnp.where(x >= 0, x, 0.01 * x)


def _prep_body(feat, btn_w, btn_b, tpw1, tpb1, tw1, tpw2, tpb2, tw2,
               f_o, sq_o, p1_o, q1_o, p2_o, q2_o):
    f = jnp.dot(feat[...], btn_w[...], preferred_element_type=jnp.float32) + btn_b[...]
    f_o[...] = f
    sq_o[...] = jnp.sum(f * f, axis=1, keepdims=True)
    p1_o[...] = jnp.dot(f, tpw1[...], preferred_element_type=jnp.float32) + tpb1[...]
    q1_o[...] = jnp.dot(f, tw1[...], preferred_element_type=jnp.float32)
    p2_o[...] = jnp.dot(f, tpw2[...], preferred_element_type=jnp.float32) + tpb2[...]
    q2_o[...] = jnp.dot(f, tw2[...], preferred_element_type=jnp.float32)


def _prep(feature, btn_w, btn_b, tpw1, tpb1, tw1, tpw2, tpb2, tw2):
    nb = N // R_PREP
    full = lambda shape: pl.BlockSpec(shape, lambda i: tuple(0 for _ in shape))
    out_shapes = (
        jax.ShapeDtypeStruct((N, D), jnp.float32),
        jax.ShapeDtypeStruct((N, 1), jnp.float32),
        jax.ShapeDtypeStruct((N, D), jnp.float32),
        jax.ShapeDtypeStruct((N, D), jnp.float32),
        jax.ShapeDtypeStruct((N, D), jnp.float32),
        jax.ShapeDtypeStruct((N, D), jnp.float32),
    )
    blk_nd = pl.BlockSpec((R_PREP, D), lambda i: (i, 0))
    return pl.pallas_call(
        _prep_body,
        grid=(nb,),
        in_specs=[
            pl.BlockSpec((R_PREP, IN_FEATS), lambda i: (i, 0)),
            full((IN_FEATS, D)), full((1, D)),
            full((D, D)), full((1, D)), full((D, D)),
            full((D, D)), full((1, D)), full((D, D)),
        ],
        out_specs=(
            blk_nd, pl.BlockSpec((R_PREP, 1), lambda i: (i, 0)),
            blk_nd, blk_nd, blk_nd, blk_nd,
        ),
        out_shape=out_shapes,
    )(feature, btn_w, btn_b, tpw1, tpb1, tw1, tpw2, tpb2, tw2)


NLANE = 128
NCHUNK = (N + NLANE - 1) // NLANE
B_INS = 6


def _knn_body(f_blk, sq_blk, f_full, sqT, idx_o, dscr):
    R = R_KNN
    g = lax.dot_general(f_blk[...], f_full[...],
                        (((1,), (1,)), ((), ())),
                        preferred_element_type=jnp.float32)
    dist = (sq_blk[...] + sqT[...]) - 2.0 * g
    big = jnp.int32(2 ** 30)
    inf = jnp.float32(jnp.inf)

    pad = NCHUNK * NLANE - N
    dscr[...] = jnp.concatenate(
        [dist, jnp.full((R, pad), inf, dtype=jnp.float32)], axis=1)

    lane = lax.broadcasted_iota(jnp.int32, (R, NLANE), 1)

    def lex_lt(av, ai, bv, bi):
        return (av < bv) | ((av == bv) & (ai < bi))

    def ins_step(c, st):
        x = dscr[:, pl.ds(c * NLANE, NLANE)]
        xi = lane + c * NLANE
        cv, ci = x, xi
        out = []
        for b in range(B_INS):
            vb, ib = st[2 * b], st[2 * b + 1]
            lt = lex_lt(cv, ci, vb, ib)
            nv = jnp.where(lt, cv, vb)
            ni = jnp.where(lt, ci, ib)
            cv, ci = jnp.where(lt, vb, cv), jnp.where(lt, ib, ci)
            out += [nv, ni]
        return tuple(out)

    init = []
    for b in range(B_INS):
        init += [jnp.full((R, NLANE), inf, dtype=jnp.float32),
                 jnp.full((R, NLANE), big, dtype=jnp.int32)]
    st = lax.fori_loop(0, NCHUNK, ins_step, tuple(init), unroll=2)

    cand_v = jnp.concatenate([st[2 * b] for b in range(B_INS)], axis=1)
    cand_i = jnp.concatenate([st[2 * b + 1] for b in range(B_INS)], axis=1)

    cols = []
    work = cand_v
    m = None
    amin = None
    for t in range(K):
        m = jnp.min(work, axis=1, keepdims=True)
        amin = jnp.min(jnp.where(work == m, cand_i, big), axis=1, keepdims=True)
        cols.append(amin)
        work = jnp.where(cand_i == amin, inf, work)
    idx_fast = jnp.concatenate(cols, axis=1)

    ovf = jnp.any(lex_lt(st[2 * (B_INS - 1)], st[2 * (B_INS - 1) + 1], m, amin))

    def slow():
        iota = lax.broadcasted_iota(jnp.int32, (R, N), 1)
        d = dist
        cs = []
        for t in range(K):
            mm = jnp.min(d, axis=1, keepdims=True)
            am = jnp.min(jnp.where(d == mm, iota, big), axis=1, keepdims=True)
            cs.append(am)
            d = jnp.where(iota == am, inf, d)
        return jnp.concatenate(cs, axis=1)

    idx_o[...] = lax.cond(ovf, slow, lambda: idx_fast)


def _knn(f, sq, sqT):
    nb = N // R_KNN
    return pl.pallas_call(
        _knn_body,
        grid=(nb,),
        in_specs=[
            pl.BlockSpec((R_KNN, D), lambda i: (i, 0)),
            pl.BlockSpec((R_KNN, 1), lambda i: (i, 0)),
            pl.BlockSpec((N, D), lambda i: (0, 0)),
            pl.BlockSpec((1, N), lambda i: (0, 0)),
        ],
        out_specs=pl.BlockSpec((R_KNN, K), lambda i: (i, 0)),
        out_shape=jax.ShapeDtypeStruct((N, K), jnp.int32),
        scratch_shapes=[pltpu.VMEM((R_KNN, NCHUNK * NLANE), jnp.float32)],
    )(f, sq, f, sqT)


NW = 32
GCH = 128


def _sc_gather(table, idx, B):
    bpw = B // NW
    nch = bpw // GCH
    mesh = plsc.VectorSubcoreMesh(core_axis_name="c", subcore_axis_name="s")

    @functools.partial(
        pl.kernel, mesh=mesh,
        compiler_params=pltpu.CompilerParams(use_tc_tiling_on_sc=False),
        out_type=jax.ShapeDtypeStruct((B, D), jnp.float32),
        scratch_types=[
            pltpu.VMEM((bpw,), jnp.int32),
            pltpu.VMEM((GCH, D), jnp.float32),
            pltpu.VMEM((GCH, D), jnp.float32),
            pltpu.SemaphoreType.DMA,
            pltpu.SemaphoreType.DMA,
        ],
    )
    def k(table_hbm, idx_hbm, out_hbm, idx_v, buf0, buf1, sem0, sem1):
        wid = lax.axis_index("s") * 2 + lax.axis_index("c")
        base = wid * bpw
        pltpu.sync_copy(idx_hbm.at[pl.ds(base, bpw)], idx_v)

        def start(g, buf, sem):
            pltpu.async_copy(table_hbm.at[idx_v.at[pl.ds(g * GCH, GCH)]], buf, sem)

        def wait(buf, sem):
            pltpu.make_async_copy(table_hbm.at[pl.ds(0, GCH)], buf, sem).wait()

        start(0, buf0, sem0)

        def body2(h, _):
            g0 = 2 * h
            start(g0 + 1, buf1, sem1)
            wait(buf0, sem0)
            pltpu.sync_copy(buf0, out_hbm.at[pl.ds(base + g0 * GCH, GCH)])

            @pl.when(g0 + 2 < nch)
            def _():
                start(g0 + 2, buf0, sem0)

            wait(buf1, sem1)
            pltpu.sync_copy(buf1, out_hbm.at[pl.ds(base + (g0 + 1) * GCH, GCH)])
            return 0

        lax.fori_loop(0, nch // 2, body2, 0)

    return k(table, idx)


def _gather_all(q1, q2, f, idx20, idx40d, idx9):
    table = jnp.concatenate([q1, q2, f], axis=0)
    idxcat = jnp.concatenate([idx20, idx40d + N, idx9 + 2 * N])
    ntot = idxcat.shape[0]
    B = ((ntot + NW * GCH - 1) // (NW * GCH)) * (NW * GCH)
    idxcat = jnp.concatenate(
        [idxcat, jnp.zeros((B - ntot,), dtype=idxcat.dtype)])
    g = _sc_gather(table, idxcat.astype(jnp.int32), B)
    n20 = idx20.shape[0]
    n40 = idx40d.shape[0]
    n9 = idx9.shape[0]
    return g[:n20], g[n20:n20 + n40], g[n20 + n40:n20 + n40 + n9]


def _edge_body(q1g, q2g, f9, p1, p2, feat,
               w11, b11, w12, b12, w21, b21, w22, b22,
               lw1, lw2, lw3, lin_b, out_o):
    er20 = lax.broadcasted_iota(jnp.int32, (R_EDGE * 20, R_EDGE), 0) // 20
    ec20 = lax.broadcasted_iota(jnp.int32, (R_EDGE * 20, R_EDGE), 1)
    e20 = (er20 == ec20).astype(jnp.float32)

    def conv(qg, p, w1, b1, w2, b2):
        pexp = jnp.dot(e20, p[...], preferred_element_type=jnp.float32)
        h = _leaky(pexp - qg[...])
        h = _leaky(jnp.dot(h, w1[...], preferred_element_type=jnp.float32) + b1[...])
        h = jnp.dot(h, w2[...], preferred_element_type=jnp.float32) + b2[...]
        return jnp.max(h.reshape(R_EDGE, 20, D), axis=1)

    h1 = conv(q1g, p1, w11, b11, w12, b12)
    h2 = conv(q2g, p2, w21, b21, w22, b22)
    hm = jnp.max(f9[...].reshape(R_EDGE, 9, D), axis=1)
    out = (jnp.dot(h1, lw1[...], preferred_element_type=jnp.float32)
           + jnp.dot(h2, lw2[...], preferred_element_type=jnp.float32)
           + jnp.dot(hm, lw3[...], preferred_element_type=jnp.float32)
           + lin_b[...] + feat[...])
    out_o[...] = out


def _edge(q1g, q2g, f9, p1, p2, feat,
          w11, b11, w12, b12, w21, b21, w22, b22, lw1, lw2, lw3, lin_b):
    nb = N // R_EDGE
    full = lambda shape: pl.BlockSpec(shape, lambda i: tuple(0 for _ in shape))
    return pl.pallas_call(
        _edge_body,
        grid=(nb,),
        in_specs=[
            pl.BlockSpec((R_EDGE * 20, D), lambda i: (i, 0)),
            pl.BlockSpec((R_EDGE * 20, D), lambda i: (i, 0)),
            pl.BlockSpec((R_EDGE * 9, D), lambda i: (i, 0)),
            pl.BlockSpec((R_EDGE, D), lambda i: (i, 0)),
            pl.BlockSpec((R_EDGE, D), lambda i: (i, 0)),
            pl.BlockSpec((R_EDGE, IN_FEATS), lambda i: (i, 0)),
            full((D, HID)), full((1, HID)), full((HID, D)), full((1, D)),
            full((D, HID)), full((1, HID)), full((HID, D)), full((1, D)),
            full((D, IN_FEATS)), full((D, IN_FEATS)), full((D, IN_FEATS)),
            full((1, IN_FEATS)),
        ],
        out_specs=pl.BlockSpec((R_EDGE, IN_FEATS), lambda i: (i, 0)),
        out_shape=jax.ShapeDtypeStruct((N, IN_FEATS), jnp.float32),
    )(q1g, q2g, f9, p1, p2, feat,
      w11, b11, w12, b12, w21, b21, w22, b22, lw1, lw2, lw3, lin_b)


def kernel(feature, btn_w, btn_b,
           g1_theta_w, g1_theta_b, g1_phi_w, g1_phi_b,
           g1_mlp_w1, g1_mlp_b1, g1_mlp_w2, g1_mlp_b2,
           g2_theta_w, g2_theta_b, g2_phi_w, g2_phi_b,
           g2_mlp_w1, g2_mlp_b1, g2_mlp_w2, g2_mlp_b2,
           lin_w, lin_b):
    r = lambda b: b.reshape(1, -1)
    tpw1 = g1_theta_w + g1_phi_w
    tpb1 = r(g1_theta_b + g1_phi_b)
    tpw2 = g2_theta_w + g2_phi_w
    tpb2 = r(g2_theta_b + g2_phi_b)

    f, sq, p1, q1, p2, q2 = _prep(feature, btn_w, r(btn_b),
                                  tpw1, tpb1, g1_theta_w,
                                  tpw2, tpb2, g2_theta_w)
    sqT = sq.reshape(1, N)
    idx = _knn(f, sq, sqT)

    idx20 = idx[:, :20].reshape(-1)
    idx40d = idx[:, ::2].reshape(-1)
    idx9 = idx[:, :9].reshape(-1)
    q1g, q2g, f9 = _gather_all(q1, q2, f, idx20, idx40d, idx9)

    out = _edge(q1g, q2g, f9, p1, p2, feature,
                g1_mlp_w1, r(g1_mlp_b1), g1_mlp_w2, r(g1_mlp_b2),
                g2_mlp_w1, r(g2_mlp_b1), g2_mlp_w2, r(g2_mlp_b2),
                lin_w[0:D, :], lin_w[D:2 * D, :], lin_w[2 * D:3 * D, :],
                r(lin_b))
    return out


# speedup vs baseline: 15.5809x; 1.1273x over previous
import functools
import jax
import jax.numpy as jnp
from jax import lax
from jax.experimental import pallas as pl
from jax.experimental.pallas import tpu as pltpu
from jax.experimental.pallas import tpu_sc as plsc

N = 10000
IN_FEATS = 256
D = 64
HID = 64
K = 40

R_PREP = 1000
R_KNN = 200
R_EDGE = 200


def _leaky(x):
    return jnp.where(x >= 0, x, 0.01 * x)


def _prep_body(feat, btn_w, btn_b, tpw1, tpb1, tw1, tpw2, tpb2, tw2,
               f_o, sq_o, p1_o, q1_o, p2_o, q2_o):
    f = jnp.dot(feat[...], btn_w[...], preferred_element_type=jnp.float32) + btn_b[...]
    f_o[...] = f
    sq_o[...] = jnp.sum(f * f, axis=1, keepdims=True)
    p1_o[...] = jnp.dot(f, tpw1[...], preferred_element_type=jnp.float32) + tpb1[...]
    q1_o[...] = jnp.dot(f, tw1[...], preferred_element_type=jnp.float32)
    p2_o[...] = jnp.dot(f, tpw2[...], preferred_element_type=jnp.float32) + tpb2[...]
    q2_o[...] = jnp.dot(f, tw2[...], preferred_element_type=jnp.float32)


def _prep(feature, btn_w, btn_b, tpw1, tpb1, tw1, tpw2, tpb2, tw2):
    nb = N // R_PREP
    full = lambda shape: pl.BlockSpec(shape, lambda i: tuple(0 for _ in shape))
    out_shapes = (
        jax.ShapeDtypeStruct((N, D), jnp.float32),
        jax.ShapeDtypeStruct((N, 1), jnp.float32),
        jax.ShapeDtypeStruct((N, D), jnp.float32),
        jax.ShapeDtypeStruct((N, D), jnp.float32),
        jax.ShapeDtypeStruct((N, D), jnp.float32),
        jax.ShapeDtypeStruct((N, D), jnp.float32),
    )
    blk_nd = pl.BlockSpec((R_PREP, D), lambda i: (i, 0))
    return pl.pallas_call(
        _prep_body,
        grid=(nb,),
        in_specs=[
            pl.BlockSpec((R_PREP, IN_FEATS), lambda i: (i, 0)),
            full((IN_FEATS, D)), full((1, D)),
            full((D, D)), full((1, D)), full((D, D)),
            full((D, D)), full((1, D)), full((D, D)),
        ],
        out_specs=(
            blk_nd, pl.BlockSpec((R_PREP, 1), lambda i: (i, 0)),
            blk_nd, blk_nd, blk_nd, blk_nd,
        ),
        out_shape=out_shapes,
    )(feature, btn_w, btn_b, tpw1, tpb1, tw1, tpw2, tpb2, tw2)


NLANE = 128
NCHUNK = (N + NLANE - 1) // NLANE
B_INS = 6


def _knn_body(f_blk, sq_blk, f_full, sqT, idx_o, dscr):
    R = R_KNN
    g = lax.dot_general(f_blk[...], f_full[...],
                        (((1,), (1,)), ((), ())),
                        preferred_element_type=jnp.float32)
    dist = (sq_blk[...] + sqT[...]) - 2.0 * g
    big = jnp.int32(2 ** 30)
    inf = jnp.float32(jnp.inf)

    pad = NCHUNK * NLANE - N
    dscr[...] = jnp.concatenate(
        [dist, jnp.full((R, pad), inf, dtype=jnp.float32)], axis=1)

    lane = lax.broadcasted_iota(jnp.int32, (R, NLANE), 1)

    def lex_lt(av, ai, bv, bi):
        return (av < bv) | ((av == bv) & (ai < bi))

    def ins_step(c, st):
        x = dscr[:, pl.ds(c * NLANE, NLANE)]
        xi = lane + c * NLANE
        cv, ci = x, xi
        inserted = None
        out = []
        for b in range(B_INS):
            vb, ib = st[2 * b], st[2 * b + 1]
            lt = cv < vb
            if inserted is not None:
                lt = inserted | lt
            nv = jnp.where(lt, cv, vb)
            ni = jnp.where(lt, ci, ib)
            cv, ci = jnp.where(lt, vb, cv), jnp.where(lt, ib, ci)
            inserted = lt
            out += [nv, ni]
        return tuple(out)

    init = []
    for b in range(B_INS):
        init += [jnp.full((R, NLANE), inf, dtype=jnp.float32),
                 jnp.full((R, NLANE), big, dtype=jnp.int32)]
    st = lax.fori_loop(0, NCHUNK, ins_step, tuple(init), unroll=2)

    cand_v = jnp.concatenate([st[2 * b] for b in range(B_INS)], axis=1)
    cand_i = jnp.concatenate([st[2 * b + 1] for b in range(B_INS)], axis=1)

    cols = []
    work = cand_v
    m = None
    amin = None
    for t in range(K):
        m = jnp.min(work, axis=1, keepdims=True)
        amin = jnp.min(jnp.where(work == m, cand_i, big), axis=1, keepdims=True)
        cols.append(amin)
        work = jnp.where(cand_i == amin, inf, work)
    idx_fast = jnp.concatenate(cols, axis=1)

    ovf = jnp.any(lex_lt(st[2 * (B_INS - 1)], st[2 * (B_INS - 1) + 1], m, amin))

    def slow():
        iota = lax.broadcasted_iota(jnp.int32, (R, N), 1)
        d = dist
        cs = []
        for t in range(K):
            mm = jnp.min(d, axis=1, keepdims=True)
            am = jnp.min(jnp.where(d == mm, iota, big), axis=1, keepdims=True)
            cs.append(am)
            d = jnp.where(iota == am, inf, d)
        return jnp.concatenate(cs, axis=1)

    idx_o[...] = lax.cond(ovf, slow, lambda: idx_fast)


def _knn(f, sq, sqT):
    nb = N // R_KNN
    return pl.pallas_call(
        _knn_body,
        grid=(nb,),
        in_specs=[
            pl.BlockSpec((R_KNN, D), lambda i: (i, 0)),
            pl.BlockSpec((R_KNN, 1), lambda i: (i, 0)),
            pl.BlockSpec((N, D), lambda i: (0, 0)),
            pl.BlockSpec((1, N), lambda i: (0, 0)),
        ],
        out_specs=pl.BlockSpec((R_KNN, K), lambda i: (i, 0)),
        out_shape=jax.ShapeDtypeStruct((N, K), jnp.int32),
        scratch_shapes=[pltpu.VMEM((R_KNN, NCHUNK * NLANE), jnp.float32)],
    )(f, sq, f, sqT)


NW = 32
GCH = 128


def _sc_gather(table, idx, B):
    bpw = B // NW
    nch = bpw // GCH
    mesh = plsc.VectorSubcoreMesh(core_axis_name="c", subcore_axis_name="s")

    @functools.partial(
        pl.kernel, mesh=mesh,
        compiler_params=pltpu.CompilerParams(use_tc_tiling_on_sc=False),
        out_type=jax.ShapeDtypeStruct((B, D), jnp.float32),
        scratch_types=[
            pltpu.VMEM((bpw,), jnp.int32),
            pltpu.VMEM((GCH, D), jnp.float32),
            pltpu.VMEM((GCH, D), jnp.float32),
            pltpu.SemaphoreType.DMA,
            pltpu.SemaphoreType.DMA,
        ],
    )
    def k(table_hbm, idx_hbm, out_hbm, idx_v, buf0, buf1, sem0, sem1):
        wid = lax.axis_index("s") * 2 + lax.axis_index("c")
        base = wid * bpw
        pltpu.sync_copy(idx_hbm.at[pl.ds(base, bpw)], idx_v)

        def start(g, buf, sem):
            pltpu.async_copy(table_hbm.at[idx_v.at[pl.ds(g * GCH, GCH)]], buf, sem)

        def wait(buf, sem):
            pltpu.make_async_copy(table_hbm.at[pl.ds(0, GCH)], buf, sem).wait()

        start(0, buf0, sem0)

        def body2(h, _):
            g0 = 2 * h
            start(g0 + 1, buf1, sem1)
            wait(buf0, sem0)
            pltpu.sync_copy(buf0, out_hbm.at[pl.ds(base + g0 * GCH, GCH)])

            @pl.when(g0 + 2 < nch)
            def _():
                start(g0 + 2, buf0, sem0)

            wait(buf1, sem1)
            pltpu.sync_copy(buf1, out_hbm.at[pl.ds(base + (g0 + 1) * GCH, GCH)])
            return 0

        lax.fori_loop(0, nch // 2, body2, 0)

    return k(table, idx)


def _gather_all(q1, q2, f, idx20, idx40d, idx9):
    table = jnp.concatenate([q1, q2, f], axis=0)
    idxcat = jnp.concatenate([idx20, idx40d + N, idx9 + 2 * N])
    ntot = idxcat.shape[0]
    B = ((ntot + NW * GCH - 1) // (NW * GCH)) * (NW * GCH)
    idxcat = jnp.concatenate(
        [idxcat, jnp.zeros((B - ntot,), dtype=idxcat.dtype)])
    g = _sc_gather(table, idxcat.astype(jnp.int32), B)
    n20 = idx20.shape[0]
    n40 = idx40d.shape[0]
    n9 = idx9.shape[0]
    return g[:n20], g[n20:n20 + n40], g[n20 + n40:n20 + n40 + n9]


def _edge_body(q1g, q2g, f9, p1, p2, feat,
               w11, b11, w12, b12, w21, b21, w22, b22,
               lw1, lw2, lw3, lin_b, out_o):
    er20 = lax.broadcasted_iota(jnp.int32, (R_EDGE * 20, R_EDGE), 0) // 20
    ec20 = lax.broadcasted_iota(jnp.int32, (R_EDGE * 20, R_EDGE), 1)
    e20 = (er20 == ec20).astype(jnp.float32)

    def conv(qg, p, w1, b1, w2, b2):
        pexp = jnp.dot(e20, p[...], preferred_element_type=jnp.float32)
        h = _leaky(pexp - qg[...])
        h = _leaky(jnp.dot(h, w1[...], preferred_element_type=jnp.float32) + b1[...])
        h = jnp.dot(h, w2[...], preferred_element_type=jnp.float32) + b2[...]
        return jnp.max(h.reshape(R_EDGE, 20, D), axis=1)

    h1 = conv(q1g, p1, w11, b11, w12, b12)
    h2 = conv(q2g, p2, w21, b21, w22, b22)
    hm = jnp.max(f9[...].reshape(R_EDGE, 9, D), axis=1)
    out = (jnp.dot(h1, lw1[...], preferred_element_type=jnp.float32)
           + jnp.dot(h2, lw2[...], preferred_element_type=jnp.float32)
           + jnp.dot(hm, lw3[...], preferred_element_type=jnp.float32)
           + lin_b[...] + feat[...])
    out_o[...] = out


def _edge(q1g, q2g, f9, p1, p2, feat,
          w11, b11, w12, b12, w21, b21, w22, b22, lw1, lw2, lw3, lin_b):
    nb = N // R_EDGE
    full = lambda shape: pl.BlockSpec(shape, lambda i: tuple(0 for _ in shape))
    return pl.pallas_call(
        _edge_body,
        grid=(nb,),
        in_specs=[
            pl.BlockSpec((R_EDGE * 20, D), lambda i: (i, 0)),
            pl.BlockSpec((R_EDGE * 20, D), lambda i: (i, 0)),
            pl.BlockSpec((R_EDGE * 9, D), lambda i: (i, 0)),
            pl.BlockSpec((R_EDGE, D), lambda i: (i, 0)),
            pl.BlockSpec((R_EDGE, D), lambda i: (i, 0)),
            pl.BlockSpec((R_EDGE, IN_FEATS), lambda i: (i, 0)),
            full((D, HID)), full((1, HID)), full((HID, D)), full((1, D)),
            full((D, HID)), full((1, HID)), full((HID, D)), full((1, D)),
            full((D, IN_FEATS)), full((D, IN_FEATS)), full((D, IN_FEATS)),
            full((1, IN_FEATS)),
        ],
        out_specs=pl.BlockSpec((R_EDGE, IN_FEATS), lambda i: (i, 0)),
        out_shape=jax.ShapeDtypeStruct((N, IN_FEATS), jnp.float32),
    )(q1g, q2g, f9, p1, p2, feat,
      w11, b11, w12, b12, w21, b21, w22, b22, lw1, lw2, lw3, lin_b)


def kernel(feature, btn_w, btn_b,
           g1_theta_w, g1_theta_b, g1_phi_w, g1_phi_b,
           g1_mlp_w1, g1_mlp_b1, g1_mlp_w2, g1_mlp_b2,
           g2_theta_w, g2_theta_b, g2_phi_w, g2_phi_b,
           g2_mlp_w1, g2_mlp_b1, g2_mlp_w2, g2_mlp_b2,
           lin_w, lin_b):
    r = lambda b: b.reshape(1, -1)
    tpw1 = g1_theta_w + g1_phi_w
    tpb1 = r(g1_theta_b + g1_phi_b)
    tpw2 = g2_theta_w + g2_phi_w
    tpb2 = r(g2_theta_b + g2_phi_b)

    f, sq, p1, q1, p2, q2 = _prep(feature, btn_w, r(btn_b),
                                  tpw1, tpb1, g1_theta_w,
                                  tpw2, tpb2, g2_theta_w)
    sqT = sq.reshape(1, N)
    idx = _knn(f, sq, sqT)

    idx20 = idx[:, :20].reshape(-1)
    idx40d = idx[:, ::2].reshape(-1)
    idx9 = idx[:, :9].reshape(-1)
    q1g, q2g, f9 = _gather_all(q1, q2, f, idx20, idx40d, idx9)

    out = _edge(q1g, q2g, f9, p1, p2, feature,
                g1_mlp_w1, r(g1_mlp_b1), g1_mlp_w2, r(g1_mlp_b2),
                g2_mlp_w1, r(g2_mlp_b1), g2_mlp_w2, r(g2_mlp_b2),
                lin_w[0:D, :], lin_w[D:2 * D, :], lin_w[2 * D:3 * D, :],
                r(lin_b))
    return out


# speedup vs baseline: 15.6244x; 1.0028x over previous
import functools
import jax
import jax.numpy as jnp
from jax import lax
from jax.experimental import pallas as pl
from jax.experimental.pallas import tpu as pltpu
from jax.experimental.pallas import tpu_sc as plsc

N = 10000
IN_FEATS = 256
D = 64
HID = 64
K = 40

R_PREP = 1000
R_KNN = 200
R_EDGE = 200


def _leaky(x):
    return jnp.where(x >= 0, x, 0.01 * x)


def _prep_body(feat, btn_w, btn_b, tpw1, tpb1, tw1, tpw2, tpb2, tw2,
               f_o, sq_o, p1_o, q1_o, p2_o, q2_o):
    f = jnp.dot(feat[...], btn_w[...], preferred_element_type=jnp.float32) + btn_b[...]
    f_o[...] = f
    sq_o[...] = jnp.sum(f * f, axis=1, keepdims=True)
    p1_o[...] = jnp.dot(f, tpw1[...], preferred_element_type=jnp.float32) + tpb1[...]
    q1_o[...] = jnp.dot(f, tw1[...], preferred_element_type=jnp.float32)
    p2_o[...] = jnp.dot(f, tpw2[...], preferred_element_type=jnp.float32) + tpb2[...]
    q2_o[...] = jnp.dot(f, tw2[...], preferred_element_type=jnp.float32)


def _prep(feature, btn_w, btn_b, tpw1, tpb1, tw1, tpw2, tpb2, tw2):
    nb = N // R_PREP
    full = lambda shape: pl.BlockSpec(shape, lambda i: tuple(0 for _ in shape))
    out_shapes = (
        jax.ShapeDtypeStruct((N, D), jnp.float32),
        jax.ShapeDtypeStruct((N, 1), jnp.float32),
        jax.ShapeDtypeStruct((N, D), jnp.float32),
        jax.ShapeDtypeStruct((N, D), jnp.float32),
        jax.ShapeDtypeStruct((N, D), jnp.float32),
        jax.ShapeDtypeStruct((N, D), jnp.float32),
    )
    blk_nd = pl.BlockSpec((R_PREP, D), lambda i: (i, 0))
    return pl.pallas_call(
        _prep_body,
        grid=(nb,),
        in_specs=[
            pl.BlockSpec((R_PREP, IN_FEATS), lambda i: (i, 0)),
            full((IN_FEATS, D)), full((1, D)),
            full((D, D)), full((1, D)), full((D, D)),
            full((D, D)), full((1, D)), full((D, D)),
        ],
        out_specs=(
            blk_nd, pl.BlockSpec((R_PREP, 1), lambda i: (i, 0)),
            blk_nd, blk_nd, blk_nd, blk_nd,
        ),
        out_shape=out_shapes,
    )(feature, btn_w, btn_b, tpw1, tpb1, tw1, tpw2, tpb2, tw2)


NLANE = 128
NCHUNK = (N + NLANE - 1) // NLANE
B_INS = 6


def _knn_body(f_blk, sq_blk, f_full, sqT, idx_o, dscr):
    R = R_KNN
    g = lax.dot_general(f_blk[...], f_full[...],
                        (((1,), (1,)), ((), ())),
                        preferred_element_type=jnp.float32)
    dist = (sq_blk[...] + sqT[...]) - 2.0 * g
    big = jnp.int32(2 ** 30)
    inf = jnp.float32(jnp.inf)

    pad = NCHUNK * NLANE - N
    dscr[...] = jnp.concatenate(
        [dist, jnp.full((R, pad), inf, dtype=jnp.float32)], axis=1)

    lane = lax.broadcasted_iota(jnp.int32, (R, NLANE), 1)

    def lex_lt(av, ai, bv, bi):
        return (av < bv) | ((av == bv) & (ai < bi))

    def ins_step(c, st):
        x = dscr[:, pl.ds(c * NLANE, NLANE)]
        xi = lane + c * NLANE
        cv, ci = x, xi
        inserted = None
        out = []
        for b in range(B_INS):
            vb, ib = st[2 * b], st[2 * b + 1]
            lt = cv < vb
            if inserted is not None:
                lt = inserted | lt
            nv = jnp.where(lt, cv, vb)
            ni = jnp.where(lt, ci, ib)
            cv, ci = jnp.where(lt, vb, cv), jnp.where(lt, ib, ci)
            inserted = lt
            out += [nv, ni]
        return tuple(out)

    init = []
    for b in range(B_INS):
        init += [jnp.full((R, NLANE), inf, dtype=jnp.float32),
                 jnp.full((R, NLANE), big, dtype=jnp.int32)]
    st = lax.fori_loop(0, NCHUNK, ins_step, tuple(init), unroll=2)

    vs = [st[2 * b] for b in range(B_INS)]
    is_ = [st[2 * b + 1] for b in range(B_INS)]
    cols = []
    m = None
    amin = None
    for t in range(K):
        m = jnp.min(vs[0], axis=1, keepdims=True)
        amin = jnp.min(jnp.where(vs[0] == m, is_[0], big), axis=1, keepdims=True)
        cols.append(amin)
        lm = is_[0] == amin
        for b in range(B_INS - 1):
            vs[b] = jnp.where(lm, vs[b + 1], vs[b])
            is_[b] = jnp.where(lm, is_[b + 1], is_[b])
        vs[B_INS - 1] = jnp.where(lm, inf, vs[B_INS - 1])
        is_[B_INS - 1] = jnp.where(lm, big, is_[B_INS - 1])
    idx_fast = jnp.concatenate(cols, axis=1)

    ovf = jnp.any(lex_lt(st[2 * (B_INS - 1)], st[2 * (B_INS - 1) + 1], m, amin))

    def slow():
        iota = lax.broadcasted_iota(jnp.int32, (R, N), 1)
        d = dist
        cs = []
        for t in range(K):
            mm = jnp.min(d, axis=1, keepdims=True)
            am = jnp.min(jnp.where(d == mm, iota, big), axis=1, keepdims=True)
            cs.append(am)
            d = jnp.where(iota == am, inf, d)
        return jnp.concatenate(cs, axis=1)

    idx_o[...] = lax.cond(ovf, slow, lambda: idx_fast)


def _knn(f, sq, sqT):
    nb = N // R_KNN
    return pl.pallas_call(
        _knn_body,
        grid=(nb,),
        in_specs=[
            pl.BlockSpec((R_KNN, D), lambda i: (i, 0)),
            pl.BlockSpec((R_KNN, 1), lambda i: (i, 0)),
            pl.BlockSpec((N, D), lambda i: (0, 0)),
            pl.BlockSpec((1, N), lambda i: (0, 0)),
        ],
        out_specs=pl.BlockSpec((R_KNN, K), lambda i: (i, 0)),
        out_shape=jax.ShapeDtypeStruct((N, K), jnp.int32),
        scratch_shapes=[pltpu.VMEM((R_KNN, NCHUNK * NLANE), jnp.float32)],
    )(f, sq, f, sqT)


NW = 32
GCH = 128


def _sc_gather(table, idx, B):
    bpw = B // NW
    nch = bpw // GCH
    mesh = plsc.VectorSubcoreMesh(core_axis_name="c", subcore_axis_name="s")

    @functools.partial(
        pl.kernel, mesh=mesh,
        compiler_params=pltpu.CompilerParams(use_tc_tiling_on_sc=False),
        out_type=jax.ShapeDtypeStruct((B, D), jnp.float32),
        scratch_types=[
            pltpu.VMEM((bpw,), jnp.int32),
            pltpu.VMEM((GCH, D), jnp.float32),
            pltpu.VMEM((GCH, D), jnp.float32),
            pltpu.SemaphoreType.DMA,
            pltpu.SemaphoreType.DMA,
        ],
    )
    def k(table_hbm, idx_hbm, out_hbm, idx_v, buf0, buf1, sem0, sem1):
        wid = lax.axis_index("s") * 2 + lax.axis_index("c")
        base = wid * bpw
        pltpu.sync_copy(idx_hbm.at[pl.ds(base, bpw)], idx_v)

        def start(g, buf, sem):
            pltpu.async_copy(table_hbm.at[idx_v.at[pl.ds(g * GCH, GCH)]], buf, sem)

        def wait(buf, sem):
            pltpu.make_async_copy(table_hbm.at[pl.ds(0, GCH)], buf, sem).wait()

        start(0, buf0, sem0)

        def body2(h, _):
            g0 = 2 * h
            start(g0 + 1, buf1, sem1)
            wait(buf0, sem0)
            pltpu.sync_copy(buf0, out_hbm.at[pl.ds(base + g0 * GCH, GCH)])

            @pl.when(g0 + 2 < nch)
            def _():
                start(g0 + 2, buf0, sem0)

            wait(buf1, sem1)
            pltpu.sync_copy(buf1, out_hbm.at[pl.ds(base + (g0 + 1) * GCH, GCH)])
            return 0

        lax.fori_loop(0, nch // 2, body2, 0)

    return k(table, idx)


def _gather_all(q1, q2, f, idx20, idx40d, idx9):
    table = jnp.concatenate([q1, q2, f], axis=0)
    idxcat = jnp.concatenate([idx20, idx40d + N, idx9 + 2 * N])
    ntot = idxcat.shape[0]
    B = ((ntot + NW * GCH - 1) // (NW * GCH)) * (NW * GCH)
    idxcat = jnp.concatenate(
        [idxcat, jnp.zeros((B - ntot,), dtype=idxcat.dtype)])
    g = _sc_gather(table, idxcat.astype(jnp.int32), B)
    n20 = idx20.shape[0]
    n40 = idx40d.shape[0]
    n9 = idx9.shape[0]
    return g[:n20], g[n20:n20 + n40], g[n20 + n40:n20 + n40 + n9]


def _edge_body(q1g, q2g, f9, p1, p2, feat,
               w11, b11, w12, b12, w21, b21, w22, b22,
               lw1, lw2, lw3, lin_b, out_o):
    er20 = lax.broadcasted_iota(jnp.int32, (R_EDGE * 20, R_EDGE), 0) // 20
    ec20 = lax.broadcasted_iota(jnp.int32, (R_EDGE * 20, R_EDGE), 1)
    e20 = (er20 == ec20).astype(jnp.float32)

    def conv(qg, p, w1, b1, w2, b2):
        pexp = jnp.dot(e20, p[...], preferred_element_type=jnp.float32)
        h = _leaky(pexp - qg[...])
        h = _leaky(jnp.dot(h, w1[...], preferred_element_type=jnp.float32) + b1[...])
        h = jnp.dot(h, w2[...], preferred_element_type=jnp.float32) + b2[...]
        return jnp.max(h.reshape(R_EDGE, 20, D), axis=1)

    h1 = conv(q1g, p1, w11, b11, w12, b12)
    h2 = conv(q2g, p2, w21, b21, w22, b22)
    hm = jnp.max(f9[...].reshape(R_EDGE, 9, D), axis=1)
    out = (jnp.dot(h1, lw1[...], preferred_element_type=jnp.float32)
           + jnp.dot(h2, lw2[...], preferred_element_type=jnp.float32)
           + jnp.dot(hm, lw3[...], preferred_element_type=jnp.float32)
           + lin_b[...] + feat[...])
    out_o[...] = out


def _edge(q1g, q2g, f9, p1, p2, feat,
          w11, b11, w12, b12, w21, b21, w22, b22, lw1, lw2, lw3, lin_b):
    nb = N // R_EDGE
    full = lambda shape: pl.BlockSpec(shape, lambda i: tuple(0 for _ in shape))
    return pl.pallas_call(
        _edge_body,
        grid=(nb,),
        in_specs=[
            pl.BlockSpec((R_EDGE * 20, D), lambda i: (i, 0)),
            pl.BlockSpec((R_EDGE * 20, D), lambda i: (i, 0)),
            pl.BlockSpec((R_EDGE * 9, D), lambda i: (i, 0)),
            pl.BlockSpec((R_EDGE, D), lambda i: (i, 0)),
            pl.BlockSpec((R_EDGE, D), lambda i: (i, 0)),
            pl.BlockSpec((R_EDGE, IN_FEATS), lambda i: (i, 0)),
            full((D, HID)), full((1, HID)), full((HID, D)), full((1, D)),
            full((D, HID)), full((1, HID)), full((HID, D)), full((1, D)),
            full((D, IN_FEATS)), full((D, IN_FEATS)), full((D, IN_FEATS)),
            full((1, IN_FEATS)),
        ],
        out_specs=pl.BlockSpec((R_EDGE, IN_FEATS), lambda i: (i, 0)),
        out_shape=jax.ShapeDtypeStruct((N, IN_FEATS), jnp.float32),
    )(q1g, q2g, f9, p1, p2, feat,
      w11, b11, w12, b12, w21, b21, w22, b22, lw1, lw2, lw3, lin_b)


def kernel(feature, btn_w, btn_b,
           g1_theta_w, g1_theta_b, g1_phi_w, g1_phi_b,
           g1_mlp_w1, g1_mlp_b1, g1_mlp_w2, g1_mlp_b2,
           g2_theta_w, g2_theta_b, g2_phi_w, g2_phi_b,
           g2_mlp_w1, g2_mlp_b1, g2_mlp_w2, g2_mlp_b2,
           lin_w, lin_b):
    r = lambda b: b.reshape(1, -1)
    tpw1 = g1_theta_w + g1_phi_w
    tpb1 = r(g1_theta_b + g1_phi_b)
    tpw2 = g2_theta_w + g2_phi_w
    tpb2 = r(g2_theta_b + g2_phi_b)

    f, sq, p1, q1, p2, q2 = _prep(feature, btn_w, r(btn_b),
                                  tpw1, tpb1, g1_theta_w,
                                  tpw2, tpb2, g2_theta_w)
    sqT = sq.reshape(1, N)
    idx = _knn(f, sq, sqT)

    idx20 = idx[:, :20].reshape(-1)
    idx40d = idx[:, ::2].reshape(-1)
    idx9 = idx[:, :9].reshape(-1)
    q1g, q2g, f9 = _gather_all(q1, q2, f, idx20, idx40d, idx9)

    out = _edge(q1g, q2g, f9, p1, p2, feature,
                g1_mlp_w1, r(g1_mlp_b1), g1_mlp_w2, r(g1_mlp_b2),
                g2_mlp_w1, r(g2_mlp_b1), g2_mlp_w2, r(g2_mlp_b2),
                lin_w[0:D, :], lin_w[D:2 * D, :], lin_w[2 * D:3 * D, :],
                r(lin_b))
    return out


# speedup vs baseline: 16.1841x; 1.0358x over previous
import functools
import jax
import jax.numpy as jnp
from jax import lax
from jax.experimental import pallas as pl
from jax.experimental.pallas import tpu as pltpu
from jax.experimental.pallas import tpu_sc as plsc

N = 10000
IN_FEATS = 256
D = 64
HID = 64
K = 40

R_PREP = 1000
R_KNN = 200
R_EDGE = 200


def _leaky(x):
    return jnp.where(x >= 0, x, 0.01 * x)


def _prep_body(feat, btn_w, btn_b, tpw1, tpb1, tw1, tpw2, tpb2, tw2,
               f_o, sq_o, p1_o, q1_o, p2_o, q2_o):
    f = jnp.dot(feat[...], btn_w[...], preferred_element_type=jnp.float32) + btn_b[...]
    f_o[...] = f
    sq_o[...] = jnp.sum(f * f, axis=1, keepdims=True)
    p1_o[...] = jnp.dot(f, tpw1[...], preferred_element_type=jnp.float32) + tpb1[...]
    q1_o[...] = jnp.dot(f, tw1[...], preferred_element_type=jnp.float32)
    p2_o[...] = jnp.dot(f, tpw2[...], preferred_element_type=jnp.float32) + tpb2[...]
    q2_o[...] = jnp.dot(f, tw2[...], preferred_element_type=jnp.float32)


def _prep(feature, btn_w, btn_b, tpw1, tpb1, tw1, tpw2, tpb2, tw2):
    nb = N // R_PREP
    full = lambda shape: pl.BlockSpec(shape, lambda i: tuple(0 for _ in shape))
    out_shapes = (
        jax.ShapeDtypeStruct((N, D), jnp.float32),
        jax.ShapeDtypeStruct((N, 1), jnp.float32),
        jax.ShapeDtypeStruct((N, D), jnp.float32),
        jax.ShapeDtypeStruct((N, D), jnp.float32),
        jax.ShapeDtypeStruct((N, D), jnp.float32),
        jax.ShapeDtypeStruct((N, D), jnp.float32),
    )
    blk_nd = pl.BlockSpec((R_PREP, D), lambda i: (i, 0))
    return pl.pallas_call(
        _prep_body,
        grid=(nb,),
        in_specs=[
            pl.BlockSpec((R_PREP, IN_FEATS), lambda i: (i, 0)),
            full((IN_FEATS, D)), full((1, D)),
            full((D, D)), full((1, D)), full((D, D)),
            full((D, D)), full((1, D)), full((D, D)),
        ],
        out_specs=(
            blk_nd, pl.BlockSpec((R_PREP, 1), lambda i: (i, 0)),
            blk_nd, blk_nd, blk_nd, blk_nd,
        ),
        out_shape=out_shapes,
    )(feature, btn_w, btn_b, tpw1, tpb1, tw1, tpw2, tpb2, tw2)


NLANE = 128
NCHUNK = (N + NLANE - 1) // NLANE
B_INS = 6


def _knn_body(f_blk, sq_blk, f_full, sqT, idx_o, dscr):
    R = R_KNN
    g = lax.dot_general(f_blk[...], f_full[...],
                        (((1,), (1,)), ((), ())),
                        preferred_element_type=jnp.float32)
    dist = (sq_blk[...] + sqT[...]) - 2.0 * g
    big = jnp.int32(2 ** 30)
    inf = jnp.float32(jnp.inf)

    pad = NCHUNK * NLANE - N
    dscr[...] = jnp.concatenate(
        [dist, jnp.full((R, pad), inf, dtype=jnp.float32)], axis=1)

    lane = lax.broadcasted_iota(jnp.int32, (R, NLANE), 1)

    def lex_lt(av, ai, bv, bi):
        return (av < bv) | ((av == bv) & (ai < bi))

    def ins_step(c, st):
        x = dscr[:, pl.ds(c * NLANE, NLANE)]
        xi = lane + c * NLANE
        cv, ci = x, xi
        inserted = None
        out = []
        for b in range(B_INS):
            vb, ib = st[2 * b], st[2 * b + 1]
            lt = cv < vb
            if inserted is not None:
                lt = inserted | lt
            nv = jnp.where(lt, cv, vb)
            ni = jnp.where(lt, ci, ib)
            cv, ci = jnp.where(lt, vb, cv), jnp.where(lt, ib, ci)
            inserted = lt
            out += [nv, ni]
        return tuple(out)

    init = []
    for b in range(B_INS):
        init += [jnp.full((R, NLANE), inf, dtype=jnp.float32),
                 jnp.full((R, NLANE), big, dtype=jnp.int32)]
    st = lax.fori_loop(0, NCHUNK, ins_step, tuple(init), unroll=2)

    vs = [st[2 * b] for b in range(B_INS)]
    is_ = [st[2 * b + 1] for b in range(B_INS)]
    cols = []
    m = None
    amin = None
    for t in range(K):
        m = jnp.min(vs[0], axis=1, keepdims=True)
        amin = jnp.min(jnp.where(vs[0] == m, is_[0], big), axis=1, keepdims=True)
        cols.append(amin)
        lm = is_[0] == amin
        for b in range(B_INS - 1):
            vs[b] = jnp.where(lm, vs[b + 1], vs[b])
            is_[b] = jnp.where(lm, is_[b + 1], is_[b])
        vs[B_INS - 1] = jnp.where(lm, inf, vs[B_INS - 1])
        is_[B_INS - 1] = jnp.where(lm, big, is_[B_INS - 1])
    idx_fast = jnp.concatenate(cols, axis=1)

    ovf = jnp.any(lex_lt(st[2 * (B_INS - 1)], st[2 * (B_INS - 1) + 1], m, amin))

    def slow():
        iota = lax.broadcasted_iota(jnp.int32, (R, N), 1)
        d = dist
        cs = []
        for t in range(K):
            mm = jnp.min(d, axis=1, keepdims=True)
            am = jnp.min(jnp.where(d == mm, iota, big), axis=1, keepdims=True)
            cs.append(am)
            d = jnp.where(iota == am, inf, d)
        return jnp.concatenate(cs, axis=1)

    idx_o[...] = lax.cond(ovf, slow, lambda: idx_fast)


def _knn(f, sq, sqT):
    nb = N // R_KNN
    return pl.pallas_call(
        _knn_body,
        grid=(nb,),
        in_specs=[
            pl.BlockSpec((R_KNN, D), lambda i: (i, 0)),
            pl.BlockSpec((R_KNN, 1), lambda i: (i, 0)),
            pl.BlockSpec((N, D), lambda i: (0, 0)),
            pl.BlockSpec((1, N), lambda i: (0, 0)),
        ],
        out_specs=pl.BlockSpec((R_KNN, K), lambda i: (i, 0)),
        out_shape=jax.ShapeDtypeStruct((N, K), jnp.int32),
        scratch_shapes=[pltpu.VMEM((R_KNN, NCHUNK * NLANE), jnp.float32)],
    )(f, sq, f, sqT)


NW = 32
GCH = 128


def _sc_gather(table, idx, B):
    bpw = B // NW
    nch = bpw // GCH
    mesh = plsc.VectorSubcoreMesh(core_axis_name="c", subcore_axis_name="s")

    @functools.partial(
        pl.kernel, mesh=mesh,
        compiler_params=pltpu.CompilerParams(use_tc_tiling_on_sc=False),
        out_type=jax.ShapeDtypeStruct((B, D), jnp.float32),
        scratch_types=[
            pltpu.VMEM((bpw,), jnp.int32),
            pltpu.VMEM((GCH, D), jnp.float32),
            pltpu.VMEM((GCH, D), jnp.float32),
            pltpu.SemaphoreType.DMA,
            pltpu.SemaphoreType.DMA,
        ],
    )
    def k(table_hbm, idx_hbm, out_hbm, idx_v, buf0, buf1, sem0, sem1):
        wid = lax.axis_index("s") * 2 + lax.axis_index("c")
        base = wid * bpw
        pltpu.sync_copy(idx_hbm.at[pl.ds(base, bpw)], idx_v)

        def start(g, buf, sem):
            pltpu.async_copy(table_hbm.at[idx_v.at[pl.ds(g * GCH, GCH)]], buf, sem)

        def wait(buf, sem):
            pltpu.make_async_copy(table_hbm.at[pl.ds(0, GCH)], buf, sem).wait()

        start(0, buf0, sem0)

        def body2(h, _):
            g0 = 2 * h
            start(g0 + 1, buf1, sem1)
            wait(buf0, sem0)
            pltpu.sync_copy(buf0, out_hbm.at[pl.ds(base + g0 * GCH, GCH)])

            @pl.when(g0 + 2 < nch)
            def _():
                start(g0 + 2, buf0, sem0)

            wait(buf1, sem1)
            pltpu.sync_copy(buf1, out_hbm.at[pl.ds(base + (g0 + 1) * GCH, GCH)])
            return 0

        lax.fori_loop(0, nch // 2, body2, 0)

    return k(table, idx)


def _gather_all(q1, q2, f, idx20, idx40d, idx9):
    table = jnp.concatenate([q1, q2, f], axis=0)
    idxcat = jnp.concatenate([idx20, idx40d + N, idx9 + 2 * N])
    ntot = idxcat.shape[0]
    B = ((ntot + NW * GCH - 1) // (NW * GCH)) * (NW * GCH)
    idxcat = jnp.concatenate(
        [idxcat, jnp.zeros((B - ntot,), dtype=idxcat.dtype)])
    g = _sc_gather(table, idxcat.astype(jnp.int32), B)
    n20 = idx20.shape[0]
    n40 = idx40d.shape[0]
    n9 = idx9.shape[0]
    return g[:n20], g[n20:n20 + n40], g[n20 + n40:n20 + n40 + n9]


def _edge_body(q1g, q2g, f9, p12, feat, w1d, b1d, w2d, b2d, lw, lin_b, out_o):
    er20 = lax.broadcasted_iota(jnp.int32, (R_EDGE * 20, R_EDGE), 0) // 20
    ec20 = lax.broadcasted_iota(jnp.int32, (R_EDGE * 20, R_EDGE), 1)
    e20 = (er20 == ec20).astype(jnp.float32)

    q12g = jnp.concatenate([q1g[...], q2g[...]], axis=1)
    pexp = jnp.dot(e20, p12[...], preferred_element_type=jnp.float32)
    h = _leaky(pexp - q12g)
    h = _leaky(jnp.dot(h, w1d[...], preferred_element_type=jnp.float32) + b1d[...])
    h = jnp.dot(h, w2d[...], preferred_element_type=jnp.float32) + b2d[...]
    h12 = jnp.max(h.reshape(R_EDGE, 20, 2 * D), axis=1)
    hm = jnp.max(f9[...].reshape(R_EDGE, 9, D), axis=1)
    hcat = jnp.concatenate([h12, hm], axis=1)
    out = (jnp.dot(hcat, lw[...], preferred_element_type=jnp.float32)
           + lin_b[...] + feat[...])
    out_o[...] = out


def _edge(q1g, q2g, f9, p12, feat, w1d, b1d, w2d, b2d, lw, lin_b):
    nb = N // R_EDGE
    full = lambda shape: pl.BlockSpec(shape, lambda i: tuple(0 for _ in shape))
    return pl.pallas_call(
        _edge_body,
        grid=(nb,),
        in_specs=[
            pl.BlockSpec((R_EDGE * 20, D), lambda i: (i, 0)),
            pl.BlockSpec((R_EDGE * 20, D), lambda i: (i, 0)),
            pl.BlockSpec((R_EDGE * 9, D), lambda i: (i, 0)),
            pl.BlockSpec((R_EDGE, 2 * D), lambda i: (i, 0)),
            pl.BlockSpec((R_EDGE, IN_FEATS), lambda i: (i, 0)),
            full((2 * D, 2 * HID)), full((1, 2 * HID)),
            full((2 * HID, 2 * D)), full((1, 2 * D)),
            full((3 * D, IN_FEATS)), full((1, IN_FEATS)),
        ],
        out_specs=pl.BlockSpec((R_EDGE, IN_FEATS), lambda i: (i, 0)),
        out_shape=jax.ShapeDtypeStruct((N, IN_FEATS), jnp.float32),
    )(q1g, q2g, f9, p12, feat, w1d, b1d, w2d, b2d, lw, lin_b)


def kernel(feature, btn_w, btn_b,
           g1_theta_w, g1_theta_b, g1_phi_w, g1_phi_b,
           g1_mlp_w1, g1_mlp_b1, g1_mlp_w2, g1_mlp_b2,
           g2_theta_w, g2_theta_b, g2_phi_w, g2_phi_b,
           g2_mlp_w1, g2_mlp_b1, g2_mlp_w2, g2_mlp_b2,
           lin_w, lin_b):
    r = lambda b: b.reshape(1, -1)
    tpw1 = g1_theta_w + g1_phi_w
    tpb1 = r(g1_theta_b + g1_phi_b)
    tpw2 = g2_theta_w + g2_phi_w
    tpb2 = r(g2_theta_b + g2_phi_b)

    f, sq, p1, q1, p2, q2 = _prep(feature, btn_w, r(btn_b),
                                  tpw1, tpb1, g1_theta_w,
                                  tpw2, tpb2, g2_theta_w)
    sqT = sq.reshape(1, N)
    idx = _knn(f, sq, sqT)

    idx20 = idx[:, :20].reshape(-1)
    idx40d = idx[:, ::2].reshape(-1)
    idx9 = idx[:, :9].reshape(-1)
    q1g, q2g, f9 = _gather_all(q1, q2, f, idx20, idx40d, idx9)

    p12 = jnp.concatenate([p1, p2], axis=1)
    z = jnp.zeros((D, HID), jnp.float32)
    w1d = jnp.block([[g1_mlp_w1, z], [z, g2_mlp_w1]])
    b1d = jnp.concatenate([g1_mlp_b1, g2_mlp_b1]).reshape(1, -1)
    z2 = jnp.zeros((HID, D), jnp.float32)
    w2d = jnp.block([[g1_mlp_w2, z2], [z2, g2_mlp_w2]])
    b2d = jnp.concatenate([g1_mlp_b2, g2_mlp_b2]).reshape(1, -1)
    out = _edge(q1g, q2g, f9, p12, feature,
                w1d, b1d, w2d, b2d, lin_w, r(lin_b))
    return out


# speedup vs baseline: 17.2879x; 1.0682x over previous
import functools
import jax
import jax.numpy as jnp
from jax import lax
from jax.experimental import pallas as pl
from jax.experimental.pallas import tpu as pltpu
from jax.experimental.pallas import tpu_sc as plsc

N = 10000
IN_FEATS = 256
D = 64
HID = 64
K = 40

R_PREP = 1000
R_KNN = 200
R_EDGE = 200


def _leaky(x):
    return jnp.where(x >= 0, x, 0.01 * x)


def _prep_body(feat, btn_w, btn_b, tpw1, tpb1, tw1, tpw2, tpb2, tw2,
               f_o, sq_o, p1_o, q1_o, p2_o, q2_o):
    f = jnp.dot(feat[...], btn_w[...], preferred_element_type=jnp.float32) + btn_b[...]
    f_o[...] = f
    sq_o[...] = jnp.sum(f * f, axis=1, keepdims=True)
    p1_o[...] = jnp.dot(f, tpw1[...], preferred_element_type=jnp.float32) + tpb1[...]
    q1_o[...] = jnp.dot(f, tw1[...], preferred_element_type=jnp.float32)
    p2_o[...] = jnp.dot(f, tpw2[...], preferred_element_type=jnp.float32) + tpb2[...]
    q2_o[...] = jnp.dot(f, tw2[...], preferred_element_type=jnp.float32)


def _prep(feature, btn_w, btn_b, tpw1, tpb1, tw1, tpw2, tpb2, tw2):
    nb = N // R_PREP
    full = lambda shape: pl.BlockSpec(shape, lambda i: tuple(0 for _ in shape))
    out_shapes = (
        jax.ShapeDtypeStruct((N, D), jnp.float32),
        jax.ShapeDtypeStruct((N, 1), jnp.float32),
        jax.ShapeDtypeStruct((N, D), jnp.float32),
        jax.ShapeDtypeStruct((N, D), jnp.float32),
        jax.ShapeDtypeStruct((N, D), jnp.float32),
        jax.ShapeDtypeStruct((N, D), jnp.float32),
    )
    blk_nd = pl.BlockSpec((R_PREP, D), lambda i: (i, 0))
    return pl.pallas_call(
        _prep_body,
        grid=(nb,),
        in_specs=[
            pl.BlockSpec((R_PREP, IN_FEATS), lambda i: (i, 0)),
            full((IN_FEATS, D)), full((1, D)),
            full((D, D)), full((1, D)), full((D, D)),
            full((D, D)), full((1, D)), full((D, D)),
        ],
        out_specs=(
            blk_nd, pl.BlockSpec((R_PREP, 1), lambda i: (i, 0)),
            blk_nd, blk_nd, blk_nd, blk_nd,
        ),
        out_shape=out_shapes,
    )(feature, btn_w, btn_b, tpw1, tpb1, tw1, tpw2, tpb2, tw2)


NLANE = 128
NCHUNK = (N + NLANE - 1) // NLANE
B_INS = 6


def _knn_body(f_blk, sq_blk, f_full, sqT, idx_o, dscr):
    R = R_KNN
    g = lax.dot_general(f_blk[...], f_full[...],
                        (((1,), (1,)), ((), ())),
                        preferred_element_type=jnp.float32)
    dist = (sq_blk[...] + sqT[...]) - 2.0 * g
    big = jnp.int32(2 ** 30)
    inf = jnp.float32(jnp.inf)

    pad = NCHUNK * NLANE - N
    dscr[...] = jnp.concatenate(
        [dist, jnp.full((R, pad), inf, dtype=jnp.float32)], axis=1)

    lane = lax.broadcasted_iota(jnp.int32, (R, NLANE), 1)

    def lex_lt(av, ai, bv, bi):
        return (av < bv) | ((av == bv) & (ai < bi))

    def ins_step(c, st):
        x = dscr[:, pl.ds(c * NLANE, NLANE)]
        xi = lane + c * NLANE
        cv, ci = x, xi
        inserted = None
        out = []
        for b in range(B_INS):
            vb, ib = st[2 * b], st[2 * b + 1]
            lt = cv < vb
            if inserted is not None:
                lt = inserted | lt
            nv = jnp.where(lt, cv, vb)
            ni = jnp.where(lt, ci, ib)
            cv, ci = jnp.where(lt, vb, cv), jnp.where(lt, ib, ci)
            inserted = lt
            out += [nv, ni]
        return tuple(out)

    init = []
    for b in range(B_INS):
        init += [jnp.full((R, NLANE), inf, dtype=jnp.float32),
                 jnp.full((R, NLANE), big, dtype=jnp.int32)]
    st = lax.fori_loop(0, NCHUNK, ins_step, tuple(init), unroll=4)

    vs = [st[2 * b] for b in range(B_INS)]
    is_ = [st[2 * b + 1] for b in range(B_INS)]
    cols = []
    m = None
    amin = None
    for t in range(K):
        m = jnp.min(vs[0], axis=1, keepdims=True)
        amin = jnp.min(jnp.where(vs[0] == m, is_[0], big), axis=1, keepdims=True)
        cols.append(amin)
        lm = is_[0] == amin
        for b in range(B_INS - 1):
            vs[b] = jnp.where(lm, vs[b + 1], vs[b])
            is_[b] = jnp.where(lm, is_[b + 1], is_[b])
        vs[B_INS - 1] = jnp.where(lm, inf, vs[B_INS - 1])
        is_[B_INS - 1] = jnp.where(lm, big, is_[B_INS - 1])
    idx_fast = jnp.concatenate(cols, axis=1)

    ovf = jnp.any(lex_lt(st[2 * (B_INS - 1)], st[2 * (B_INS - 1) + 1], m, amin))

    def slow():
        iota = lax.broadcasted_iota(jnp.int32, (R, N), 1)
        d = dist
        cs = []
        for t in range(K):
            mm = jnp.min(d, axis=1, keepdims=True)
            am = jnp.min(jnp.where(d == mm, iota, big), axis=1, keepdims=True)
            cs.append(am)
            d = jnp.where(iota == am, inf, d)
        return jnp.concatenate(cs, axis=1)

    idx_o[...] = lax.cond(ovf, slow, lambda: idx_fast)


def _knn(f, sq, sqT):
    nb = N // R_KNN
    return pl.pallas_call(
        _knn_body,
        grid=(nb,),
        in_specs=[
            pl.BlockSpec((R_KNN, D), lambda i: (i, 0)),
            pl.BlockSpec((R_KNN, 1), lambda i: (i, 0)),
            pl.BlockSpec((N, D), lambda i: (0, 0)),
            pl.BlockSpec((1, N), lambda i: (0, 0)),
        ],
        out_specs=pl.BlockSpec((R_KNN, K), lambda i: (i, 0)),
        out_shape=jax.ShapeDtypeStruct((N, K), jnp.int32),
        scratch_shapes=[pltpu.VMEM((R_KNN, NCHUNK * NLANE), jnp.float32)],
    )(f, sq, f, sqT)


NW = 32
GCH = 128


def _sc_gather(table, idx, B):
    bpw = B // NW
    nch = bpw // GCH
    mesh = plsc.VectorSubcoreMesh(core_axis_name="c", subcore_axis_name="s")

    @functools.partial(
        pl.kernel, mesh=mesh,
        compiler_params=pltpu.CompilerParams(use_tc_tiling_on_sc=False),
        out_type=jax.ShapeDtypeStruct((B, D), jnp.float32),
        scratch_types=[
            pltpu.VMEM((bpw,), jnp.int32),
            pltpu.VMEM((GCH, D), jnp.float32),
            pltpu.VMEM((GCH, D), jnp.float32),
            pltpu.SemaphoreType.DMA,
            pltpu.SemaphoreType.DMA,
        ],
    )
    def k(table_hbm, idx_hbm, out_hbm, idx_v, buf0, buf1, sem0, sem1):
        wid = lax.axis_index("s") * 2 + lax.axis_index("c")
        base = wid * bpw
        pltpu.sync_copy(idx_hbm.at[pl.ds(base, bpw)], idx_v)

        def start(g, buf, sem):
            pltpu.async_copy(table_hbm.at[idx_v.at[pl.ds(g * GCH, GCH)]], buf, sem)

        def wait(buf, sem):
            pltpu.make_async_copy(table_hbm.at[pl.ds(0, GCH)], buf, sem).wait()

        start(0, buf0, sem0)

        def body2(h, _):
            g0 = 2 * h
            start(g0 + 1, buf1, sem1)
            wait(buf0, sem0)
            pltpu.sync_copy(buf0, out_hbm.at[pl.ds(base + g0 * GCH, GCH)])

            @pl.when(g0 + 2 < nch)
            def _():
                start(g0 + 2, buf0, sem0)

            wait(buf1, sem1)
            pltpu.sync_copy(buf1, out_hbm.at[pl.ds(base + (g0 + 1) * GCH, GCH)])
            return 0

        lax.fori_loop(0, nch // 2, body2, 0)

    return k(table, idx)


def _gather_all(q1, q2, f, idx20, idx40d, idx9):
    table = jnp.concatenate([q1, q2, f], axis=0)
    idxcat = jnp.concatenate([idx20, idx40d + N, idx9 + 2 * N])
    ntot = idxcat.shape[0]
    B = ((ntot + NW * GCH - 1) // (NW * GCH)) * (NW * GCH)
    idxcat = jnp.concatenate(
        [idxcat, jnp.zeros((B - ntot,), dtype=idxcat.dtype)])
    g = _sc_gather(table, idxcat.astype(jnp.int32), B)
    n20 = idx20.shape[0]
    n40 = idx40d.shape[0]
    n9 = idx9.shape[0]
    return g[:n20], g[n20:n20 + n40], g[n20 + n40:n20 + n40 + n9]


def _edge_body(q1g, q2g, f9, p12, feat, w1d, b1d, w2d, b2d, lw, lin_b, out_o):
    er20 = lax.broadcasted_iota(jnp.int32, (R_EDGE * 20, R_EDGE), 0) // 20
    ec20 = lax.broadcasted_iota(jnp.int32, (R_EDGE * 20, R_EDGE), 1)
    e20 = (er20 == ec20).astype(jnp.float32)

    q12g = jnp.concatenate([q1g[...], q2g[...]], axis=1)
    pexp = jnp.dot(e20, p12[...], preferred_element_type=jnp.float32)
    h = _leaky(pexp - q12g)
    h = _leaky(jnp.dot(h, w1d[...], preferred_element_type=jnp.float32) + b1d[...])
    h = jnp.dot(h, w2d[...], preferred_element_type=jnp.float32) + b2d[...]
    h12 = jnp.max(h.reshape(R_EDGE, 20, 2 * D), axis=1)
    hm = jnp.max(f9[...].reshape(R_EDGE, 9, D), axis=1)
    hcat = jnp.concatenate([h12, hm], axis=1)
    out = (jnp.dot(hcat, lw[...], preferred_element_type=jnp.float32)
           + lin_b[...] + feat[...])
    out_o[...] = out


def _edge(q1g, q2g, f9, p12, feat, w1d, b1d, w2d, b2d, lw, lin_b):
    nb = N // R_EDGE
    full = lambda shape: pl.BlockSpec(shape, lambda i: tuple(0 for _ in shape))
    return pl.pallas_call(
        _edge_body,
        grid=(nb,),
        in_specs=[
            pl.BlockSpec((R_EDGE * 20, D), lambda i: (i, 0)),
            pl.BlockSpec((R_EDGE * 20, D), lambda i: (i, 0)),
            pl.BlockSpec((R_EDGE * 9, D), lambda i: (i, 0)),
            pl.BlockSpec((R_EDGE, 2 * D), lambda i: (i, 0)),
            pl.BlockSpec((R_EDGE, IN_FEATS), lambda i: (i, 0)),
            full((2 * D, 2 * HID)), full((1, 2 * HID)),
            full((2 * HID, 2 * D)), full((1, 2 * D)),
            full((3 * D, IN_FEATS)), full((1, IN_FEATS)),
        ],
        out_specs=pl.BlockSpec((R_EDGE, IN_FEATS), lambda i: (i, 0)),
        out_shape=jax.ShapeDtypeStruct((N, IN_FEATS), jnp.float32),
    )(q1g, q2g, f9, p12, feat, w1d, b1d, w2d, b2d, lw, lin_b)


def kernel(feature, btn_w, btn_b,
           g1_theta_w, g1_theta_b, g1_phi_w, g1_phi_b,
           g1_mlp_w1, g1_mlp_b1, g1_mlp_w2, g1_mlp_b2,
           g2_theta_w, g2_theta_b, g2_phi_w, g2_phi_b,
           g2_mlp_w1, g2_mlp_b1, g2_mlp_w2, g2_mlp_b2,
           lin_w, lin_b):
    r = lambda b: b.reshape(1, -1)
    tpw1 = g1_theta_w + g1_phi_w
    tpb1 = r(g1_theta_b + g1_phi_b)
    tpw2 = g2_theta_w + g2_phi_w
    tpb2 = r(g2_theta_b + g2_phi_b)

    f, sq, p1, q1, p2, q2 = _prep(feature, btn_w, r(btn_b),
                                  tpw1, tpb1, g1_theta_w,
                                  tpw2, tpb2, g2_theta_w)
    sqT = sq.reshape(1, N)
    idx = _knn(f, sq, sqT)

    idx20 = idx[:, :20].reshape(-1)
    idx40d = idx[:, ::2].reshape(-1)
    idx9 = idx[:, :9].reshape(-1)
    q1g, q2g, f9 = _gather_all(q1, q2, f, idx20, idx40d, idx9)

    p12 = jnp.concatenate([p1, p2], axis=1)
    z = jnp.zeros((D, HID), jnp.float32)
    w1d = jnp.block([[g1_mlp_w1, z], [z, g2_mlp_w1]])
    b1d = jnp.concatenate([g1_mlp_b1, g2_mlp_b1]).reshape(1, -1)
    z2 = jnp.zeros((HID, D), jnp.float32)
    w2d = jnp.block([[g1_mlp_w2, z2], [z2, g2_mlp_w2]])
    b2d = jnp.concatenate([g1_mlp_b2, g2_mlp_b2]).reshape(1, -1)
    out = _edge(q1g, q2g, f9, p12, feature,
                w1d, b1d, w2d, b2d, lin_w, r(lin_b))
    return out


# speedup vs baseline: 17.9979x; 1.0411x over previous
import functools
import jax
import jax.numpy as jnp
from jax import lax
from jax.experimental import pallas as pl
from jax.experimental.pallas import tpu as pltpu
from jax.experimental.pallas import tpu_sc as plsc

N = 10000
IN_FEATS = 256
D = 64
HID = 64
K = 40

R_PREP = 1000
R_KNN = 200
R_EDGE = 200


def _leaky(x):
    return jnp.where(x >= 0, x, 0.01 * x)


def _prep_body(feat, btn_w, btn_b, tpw1, tpb1, tw1, tpw2, tpb2, tw2,
               f_o, sq_o, p1_o, q1_o, p2_o, q2_o):
    f = jnp.dot(feat[...], btn_w[...], preferred_element_type=jnp.float32) + btn_b[...]
    f_o[...] = f
    sq_o[...] = jnp.sum(f * f, axis=1, keepdims=True)
    p1_o[...] = jnp.dot(f, tpw1[...], preferred_element_type=jnp.float32) + tpb1[...]
    q1_o[...] = jnp.dot(f, tw1[...], preferred_element_type=jnp.float32)
    p2_o[...] = jnp.dot(f, tpw2[...], preferred_element_type=jnp.float32) + tpb2[...]
    q2_o[...] = jnp.dot(f, tw2[...], preferred_element_type=jnp.float32)


def _prep(feature, btn_w, btn_b, tpw1, tpb1, tw1, tpw2, tpb2, tw2):
    nb = N // R_PREP
    full = lambda shape: pl.BlockSpec(shape, lambda i: tuple(0 for _ in shape))
    out_shapes = (
        jax.ShapeDtypeStruct((N, D), jnp.float32),
        jax.ShapeDtypeStruct((N, 1), jnp.float32),
        jax.ShapeDtypeStruct((N, D), jnp.float32),
        jax.ShapeDtypeStruct((N, D), jnp.float32),
        jax.ShapeDtypeStruct((N, D), jnp.float32),
        jax.ShapeDtypeStruct((N, D), jnp.float32),
    )
    blk_nd = pl.BlockSpec((R_PREP, D), lambda i: (i, 0))
    return pl.pallas_call(
        _prep_body,
        grid=(nb,),
        in_specs=[
            pl.BlockSpec((R_PREP, IN_FEATS), lambda i: (i, 0)),
            full((IN_FEATS, D)), full((1, D)),
            full((D, D)), full((1, D)), full((D, D)),
            full((D, D)), full((1, D)), full((D, D)),
        ],
        out_specs=(
            blk_nd, pl.BlockSpec((R_PREP, 1), lambda i: (i, 0)),
            blk_nd, blk_nd, blk_nd, blk_nd,
        ),
        out_shape=out_shapes,
    )(feature, btn_w, btn_b, tpw1, tpb1, tw1, tpw2, tpb2, tw2)


NLANE = 128
NCHUNK = (N + NLANE - 1) // NLANE
B_INS = 6


def _knn_body(f_blk, sq_blk, f_full, sqT, idx_o, dscr):
    R = R_KNN
    g = lax.dot_general(f_blk[...], f_full[...],
                        (((1,), (1,)), ((), ())),
                        preferred_element_type=jnp.float32)
    dist = (sq_blk[...] + sqT[...]) - 2.0 * g
    big = jnp.int32(2 ** 30)
    inf = jnp.float32(jnp.inf)

    pad = NCHUNK * NLANE - N
    dscr[...] = jnp.concatenate(
        [dist, jnp.full((R, pad), inf, dtype=jnp.float32)], axis=1)

    lane = lax.broadcasted_iota(jnp.int32, (R, NLANE), 1)

    def lex_lt(av, ai, bv, bi):
        return (av < bv) | ((av == bv) & (ai < bi))

    def ins_step(c, st):
        x = dscr[:, pl.ds(c * NLANE, NLANE)]
        xi = lane + c * NLANE
        cv, ci = x, xi
        inserted = None
        out = []
        for b in range(B_INS):
            vb, ib = st[2 * b], st[2 * b + 1]
            lt = cv < vb
            if inserted is not None:
                lt = inserted | lt
            nv = jnp.where(lt, cv, vb)
            ni = jnp.where(lt, ci, ib)
            cv, ci = jnp.where(lt, vb, cv), jnp.where(lt, ib, ci)
            inserted = lt
            out += [nv, ni]
        return tuple(out)

    init = []
    for b in range(B_INS):
        init += [jnp.full((R, NLANE), inf, dtype=jnp.float32),
                 jnp.full((R, NLANE), big, dtype=jnp.int32)]
    st = lax.fori_loop(0, NCHUNK, ins_step, tuple(init), unroll=8)

    vs = [st[2 * b] for b in range(B_INS)]
    is_ = [st[2 * b + 1] for b in range(B_INS)]
    cols = []
    m = None
    amin = None
    for t in range(K):
        m = jnp.min(vs[0], axis=1, keepdims=True)
        amin = jnp.min(jnp.where(vs[0] == m, is_[0], big), axis=1, keepdims=True)
        cols.append(amin)
        lm = is_[0] == amin
        for b in range(B_INS - 1):
            vs[b] = jnp.where(lm, vs[b + 1], vs[b])
            is_[b] = jnp.where(lm, is_[b + 1], is_[b])
        vs[B_INS - 1] = jnp.where(lm, inf, vs[B_INS - 1])
        is_[B_INS - 1] = jnp.where(lm, big, is_[B_INS - 1])
    idx_fast = jnp.concatenate(cols, axis=1)

    ovf = jnp.any(lex_lt(st[2 * (B_INS - 1)], st[2 * (B_INS - 1) + 1], m, amin))

    def slow():
        iota = lax.broadcasted_iota(jnp.int32, (R, N), 1)
        d = dist
        cs = []
        for t in range(K):
            mm = jnp.min(d, axis=1, keepdims=True)
            am = jnp.min(jnp.where(d == mm, iota, big), axis=1, keepdims=True)
            cs.append(am)
            d = jnp.where(iota == am, inf, d)
        return jnp.concatenate(cs, axis=1)

    idx_o[...] = lax.cond(ovf, slow, lambda: idx_fast)


def _knn(f, sq, sqT):
    nb = N // R_KNN
    return pl.pallas_call(
        _knn_body,
        grid=(nb,),
        in_specs=[
            pl.BlockSpec((R_KNN, D), lambda i: (i, 0)),
            pl.BlockSpec((R_KNN, 1), lambda i: (i, 0)),
            pl.BlockSpec((N, D), lambda i: (0, 0)),
            pl.BlockSpec((1, N), lambda i: (0, 0)),
        ],
        out_specs=pl.BlockSpec((R_KNN, K), lambda i: (i, 0)),
        out_shape=jax.ShapeDtypeStruct((N, K), jnp.int32),
        scratch_shapes=[pltpu.VMEM((R_KNN, NCHUNK * NLANE), jnp.float32)],
    )(f, sq, f, sqT)


NW = 32
GCH = 128


def _sc_gather(table, idx, B):
    bpw = B // NW
    nch = bpw // GCH
    mesh = plsc.VectorSubcoreMesh(core_axis_name="c", subcore_axis_name="s")

    @functools.partial(
        pl.kernel, mesh=mesh,
        compiler_params=pltpu.CompilerParams(use_tc_tiling_on_sc=False),
        out_type=jax.ShapeDtypeStruct((B, D), jnp.float32),
        scratch_types=[
            pltpu.VMEM((bpw,), jnp.int32),
            pltpu.VMEM((GCH, D), jnp.float32),
            pltpu.VMEM((GCH, D), jnp.float32),
            pltpu.SemaphoreType.DMA,
            pltpu.SemaphoreType.DMA,
        ],
    )
    def k(table_hbm, idx_hbm, out_hbm, idx_v, buf0, buf1, sem0, sem1):
        wid = lax.axis_index("s") * 2 + lax.axis_index("c")
        base = wid * bpw
        pltpu.sync_copy(idx_hbm.at[pl.ds(base, bpw)], idx_v)

        def start(g, buf, sem):
            pltpu.async_copy(table_hbm.at[idx_v.at[pl.ds(g * GCH, GCH)]], buf, sem)

        def wait(buf, sem):
            pltpu.make_async_copy(table_hbm.at[pl.ds(0, GCH)], buf, sem).wait()

        start(0, buf0, sem0)

        def body2(h, _):
            g0 = 2 * h
            start(g0 + 1, buf1, sem1)
            wait(buf0, sem0)
            pltpu.sync_copy(buf0, out_hbm.at[pl.ds(base + g0 * GCH, GCH)])

            @pl.when(g0 + 2 < nch)
            def _():
                start(g0 + 2, buf0, sem0)

            wait(buf1, sem1)
            pltpu.sync_copy(buf1, out_hbm.at[pl.ds(base + (g0 + 1) * GCH, GCH)])
            return 0

        lax.fori_loop(0, nch // 2, body2, 0)

    return k(table, idx)


def _gather_all(q1, q2, f, idx20, idx40d, idx9):
    table = jnp.concatenate([q1, q2, f], axis=0)
    idxcat = jnp.concatenate([idx20, idx40d + N, idx9 + 2 * N])
    ntot = idxcat.shape[0]
    B = ((ntot + NW * GCH - 1) // (NW * GCH)) * (NW * GCH)
    idxcat = jnp.concatenate(
        [idxcat, jnp.zeros((B - ntot,), dtype=idxcat.dtype)])
    g = _sc_gather(table, idxcat.astype(jnp.int32), B)
    n20 = idx20.shape[0]
    n40 = idx40d.shape[0]
    n9 = idx9.shape[0]
    return g[:n20], g[n20:n20 + n40], g[n20 + n40:n20 + n40 + n9]


def _edge_body(q1g, q2g, f9, p12, feat, w1d, b1d, w2d, b2d, lw, lin_b, out_o):
    er20 = lax.broadcasted_iota(jnp.int32, (R_EDGE * 20, R_EDGE), 0) // 20
    ec20 = lax.broadcasted_iota(jnp.int32, (R_EDGE * 20, R_EDGE), 1)
    e20 = (er20 == ec20).astype(jnp.float32)

    q12g = jnp.concatenate([q1g[...], q2g[...]], axis=1)
    pexp = jnp.dot(e20, p12[...], preferred_element_type=jnp.float32)
    h = _leaky(pexp - q12g)
    h = _leaky(jnp.dot(h, w1d[...], preferred_element_type=jnp.float32) + b1d[...])
    h = jnp.dot(h, w2d[...], preferred_element_type=jnp.float32) + b2d[...]
    h12 = jnp.max(h.reshape(R_EDGE, 20, 2 * D), axis=1)
    hm = jnp.max(f9[...].reshape(R_EDGE, 9, D), axis=1)
    hcat = jnp.concatenate([h12, hm], axis=1)
    out = (jnp.dot(hcat, lw[...], preferred_element_type=jnp.float32)
           + lin_b[...] + feat[...])
    out_o[...] = out


def _edge(q1g, q2g, f9, p12, feat, w1d, b1d, w2d, b2d, lw, lin_b):
    nb = N // R_EDGE
    full = lambda shape: pl.BlockSpec(shape, lambda i: tuple(0 for _ in shape))
    return pl.pallas_call(
        _edge_body,
        grid=(nb,),
        in_specs=[
            pl.BlockSpec((R_EDGE * 20, D), lambda i: (i, 0)),
            pl.BlockSpec((R_EDGE * 20, D), lambda i: (i, 0)),
            pl.BlockSpec((R_EDGE * 9, D), lambda i: (i, 0)),
            pl.BlockSpec((R_EDGE, 2 * D), lambda i: (i, 0)),
            pl.BlockSpec((R_EDGE, IN_FEATS), lambda i: (i, 0)),
            full((2 * D, 2 * HID)), full((1, 2 * HID)),
            full((2 * HID, 2 * D)), full((1, 2 * D)),
            full((3 * D, IN_FEATS)), full((1, IN_FEATS)),
        ],
        out_specs=pl.BlockSpec((R_EDGE, IN_FEATS), lambda i: (i, 0)),
        out_shape=jax.ShapeDtypeStruct((N, IN_FEATS), jnp.float32),
    )(q1g, q2g, f9, p12, feat, w1d, b1d, w2d, b2d, lw, lin_b)


def kernel(feature, btn_w, btn_b,
           g1_theta_w, g1_theta_b, g1_phi_w, g1_phi_b,
           g1_mlp_w1, g1_mlp_b1, g1_mlp_w2, g1_mlp_b2,
           g2_theta_w, g2_theta_b, g2_phi_w, g2_phi_b,
           g2_mlp_w1, g2_mlp_b1, g2_mlp_w2, g2_mlp_b2,
           lin_w, lin_b):
    r = lambda b: b.reshape(1, -1)
    tpw1 = g1_theta_w + g1_phi_w
    tpb1 = r(g1_theta_b + g1_phi_b)
    tpw2 = g2_theta_w + g2_phi_w
    tpb2 = r(g2_theta_b + g2_phi_b)

    f, sq, p1, q1, p2, q2 = _prep(feature, btn_w, r(btn_b),
                                  tpw1, tpb1, g1_theta_w,
                                  tpw2, tpb2, g2_theta_w)
    sqT = sq.reshape(1, N)
    idx = _knn(f, sq, sqT)

    idx20 = idx[:, :20].reshape(-1)
    idx40d = idx[:, ::2].reshape(-1)
    idx9 = idx[:, :9].reshape(-1)
    q1g, q2g, f9 = _gather_all(q1, q2, f, idx20, idx40d, idx9)

    p12 = jnp.concatenate([p1, p2], axis=1)
    z = jnp.zeros((D, HID), jnp.float32)
    w1d = jnp.block([[g1_mlp_w1, z], [z, g2_mlp_w1]])
    b1d = jnp.concatenate([g1_mlp_b1, g2_mlp_b1]).reshape(1, -1)
    z2 = jnp.zeros((HID, D), jnp.float32)
    w2d = jnp.block([[g1_mlp_w2, z2], [z2, g2_mlp_w2]])
    b2d = jnp.concatenate([g1_mlp_b2, g2_mlp_b2]).reshape(1, -1)
    out = _edge(q1g, q2g, f9, p12, feature,
                w1d, b1d, w2d, b2d, lin_w, r(lin_b))
    return out


# speedup vs baseline: 18.2271x; 1.0127x over previous
import functools
import jax
import jax.numpy as jnp
from jax import lax
from jax.experimental import pallas as pl
from jax.experimental.pallas import tpu as pltpu
from jax.experimental.pallas import tpu_sc as plsc

N = 10000
IN_FEATS = 256
D = 64
HID = 64
K = 40

R_PREP = 1000
R_KNN = 200
R_EDGE = 200


def _leaky(x):
    return jnp.where(x >= 0, x, 0.01 * x)


def _prep_body(feat, btn_w, btn_b, tpw1, tpb1, tw1, tpw2, tpb2, tw2,
               f_o, sq_o, p1_o, q1_o, p2_o, q2_o):
    f = jnp.dot(feat[...], btn_w[...], preferred_element_type=jnp.float32) + btn_b[...]
    f_o[...] = f
    sq_o[...] = jnp.sum(f * f, axis=1, keepdims=True)
    p1_o[...] = jnp.dot(f, tpw1[...], preferred_element_type=jnp.float32) + tpb1[...]
    q1_o[...] = jnp.dot(f, tw1[...], preferred_element_type=jnp.float32)
    p2_o[...] = jnp.dot(f, tpw2[...], preferred_element_type=jnp.float32) + tpb2[...]
    q2_o[...] = jnp.dot(f, tw2[...], preferred_element_type=jnp.float32)


def _prep(feature, btn_w, btn_b, tpw1, tpb1, tw1, tpw2, tpb2, tw2):
    nb = N // R_PREP
    full = lambda shape: pl.BlockSpec(shape, lambda i: tuple(0 for _ in shape))
    out_shapes = (
        jax.ShapeDtypeStruct((N, D), jnp.float32),
        jax.ShapeDtypeStruct((N, 1), jnp.float32),
        jax.ShapeDtypeStruct((N, D), jnp.float32),
        jax.ShapeDtypeStruct((N, D), jnp.float32),
        jax.ShapeDtypeStruct((N, D), jnp.float32),
        jax.ShapeDtypeStruct((N, D), jnp.float32),
    )
    blk_nd = pl.BlockSpec((R_PREP, D), lambda i: (i, 0))
    return pl.pallas_call(
        _prep_body,
        grid=(nb,),
        in_specs=[
            pl.BlockSpec((R_PREP, IN_FEATS), lambda i: (i, 0)),
            full((IN_FEATS, D)), full((1, D)),
            full((D, D)), full((1, D)), full((D, D)),
            full((D, D)), full((1, D)), full((D, D)),
        ],
        out_specs=(
            blk_nd, pl.BlockSpec((R_PREP, 1), lambda i: (i, 0)),
            blk_nd, blk_nd, blk_nd, blk_nd,
        ),
        out_shape=out_shapes,
    )(feature, btn_w, btn_b, tpw1, tpb1, tw1, tpw2, tpb2, tw2)


NLANE = 128
NCHUNK = (N + NLANE - 1) // NLANE
B_INS = 6


def _knn_body(f_blk, sq_blk, f_full, sqT, idx_o, dscr):
    R = R_KNN
    g = lax.dot_general(f_blk[...], f_full[...],
                        (((1,), (1,)), ((), ())),
                        preferred_element_type=jnp.float32)
    dist = (sq_blk[...] + sqT[...]) - 2.0 * g
    big = jnp.int32(2 ** 30)
    inf = jnp.float32(jnp.inf)

    pad = NCHUNK * NLANE - N
    dscr[...] = jnp.concatenate(
        [dist, jnp.full((R, pad), inf, dtype=jnp.float32)], axis=1)

    lane = lax.broadcasted_iota(jnp.int32, (R, NLANE), 1)

    def lex_lt(av, ai, bv, bi):
        return (av < bv) | ((av == bv) & (ai < bi))

    def ins_step(c, st):
        x = dscr[:, pl.ds(c * NLANE, NLANE)]
        xi = lane + c * NLANE
        cv, ci = x, xi
        inserted = None
        out = []
        for b in range(B_INS):
            vb, ib = st[2 * b], st[2 * b + 1]
            lt = cv < vb
            if inserted is not None:
                lt = inserted | lt
            nv = jnp.where(lt, cv, vb)
            ni = jnp.where(lt, ci, ib)
            cv, ci = jnp.where(lt, vb, cv), jnp.where(lt, ib, ci)
            inserted = lt
            out += [nv, ni]
        return tuple(out)

    init = []
    for b in range(B_INS):
        init += [jnp.full((R, NLANE), inf, dtype=jnp.float32),
                 jnp.full((R, NLANE), big, dtype=jnp.int32)]
    st = lax.fori_loop(0, NCHUNK, ins_step, tuple(init), unroll=16)

    vs = [st[2 * b] for b in range(B_INS)]
    is_ = [st[2 * b + 1] for b in range(B_INS)]
    cols = []
    m = None
    amin = None
    for t in range(K):
        m = jnp.min(vs[0], axis=1, keepdims=True)
        amin = jnp.min(jnp.where(vs[0] == m, is_[0], big), axis=1, keepdims=True)
        cols.append(amin)
        lm = is_[0] == amin
        for b in range(B_INS - 1):
            vs[b] = jnp.where(lm, vs[b + 1], vs[b])
            is_[b] = jnp.where(lm, is_[b + 1], is_[b])
        vs[B_INS - 1] = jnp.where(lm, inf, vs[B_INS - 1])
        is_[B_INS - 1] = jnp.where(lm, big, is_[B_INS - 1])
    idx_fast = jnp.concatenate(cols, axis=1)

    ovf = jnp.any(lex_lt(st[2 * (B_INS - 1)], st[2 * (B_INS - 1) + 1], m, amin))

    def slow():
        iota = lax.broadcasted_iota(jnp.int32, (R, N), 1)
        d = dist
        cs = []
        for t in range(K):
            mm = jnp.min(d, axis=1, keepdims=True)
            am = jnp.min(jnp.where(d == mm, iota, big), axis=1, keepdims=True)
            cs.append(am)
            d = jnp.where(iota == am, inf, d)
        return jnp.concatenate(cs, axis=1)

    idx_o[...] = lax.cond(ovf, slow, lambda: idx_fast)


def _knn(f, sq, sqT):
    nb = N // R_KNN
    return pl.pallas_call(
        _knn_body,
        grid=(nb,),
        in_specs=[
            pl.BlockSpec((R_KNN, D), lambda i: (i, 0)),
            pl.BlockSpec((R_KNN, 1), lambda i: (i, 0)),
            pl.BlockSpec((N, D), lambda i: (0, 0)),
            pl.BlockSpec((1, N), lambda i: (0, 0)),
        ],
        out_specs=pl.BlockSpec((R_KNN, K), lambda i: (i, 0)),
        out_shape=jax.ShapeDtypeStruct((N, K), jnp.int32),
        scratch_shapes=[pltpu.VMEM((R_KNN, NCHUNK * NLANE), jnp.float32)],
    )(f, sq, f, sqT)


NW = 32
GCH = 128


def _sc_gather(table, idx, B):
    bpw = B // NW
    nch = bpw // GCH
    mesh = plsc.VectorSubcoreMesh(core_axis_name="c", subcore_axis_name="s")

    @functools.partial(
        pl.kernel, mesh=mesh,
        compiler_params=pltpu.CompilerParams(use_tc_tiling_on_sc=False),
        out_type=jax.ShapeDtypeStruct((B, D), jnp.float32),
        scratch_types=[
            pltpu.VMEM((bpw,), jnp.int32),
            pltpu.VMEM((GCH, D), jnp.float32),
            pltpu.VMEM((GCH, D), jnp.float32),
            pltpu.SemaphoreType.DMA,
            pltpu.SemaphoreType.DMA,
        ],
    )
    def k(table_hbm, idx_hbm, out_hbm, idx_v, buf0, buf1, sem0, sem1):
        wid = lax.axis_index("s") * 2 + lax.axis_index("c")
        base = wid * bpw
        pltpu.sync_copy(idx_hbm.at[pl.ds(base, bpw)], idx_v)

        def start(g, buf, sem):
            pltpu.async_copy(table_hbm.at[idx_v.at[pl.ds(g * GCH, GCH)]], buf, sem)

        def wait(buf, sem):
            pltpu.make_async_copy(table_hbm.at[pl.ds(0, GCH)], buf, sem).wait()

        start(0, buf0, sem0)

        def body2(h, _):
            g0 = 2 * h
            start(g0 + 1, buf1, sem1)
            wait(buf0, sem0)
            pltpu.sync_copy(buf0, out_hbm.at[pl.ds(base + g0 * GCH, GCH)])

            @pl.when(g0 + 2 < nch)
            def _():
                start(g0 + 2, buf0, sem0)

            wait(buf1, sem1)
            pltpu.sync_copy(buf1, out_hbm.at[pl.ds(base + (g0 + 1) * GCH, GCH)])
            return 0

        lax.fori_loop(0, nch // 2, body2, 0)

    return k(table, idx)


def _gather_all(q1, q2, f, idx20, idx40d, idx9):
    table = jnp.concatenate([q1, q2, f], axis=0)
    idxcat = jnp.concatenate([idx20, idx40d + N, idx9 + 2 * N])
    ntot = idxcat.shape[0]
    B = ((ntot + NW * GCH - 1) // (NW * GCH)) * (NW * GCH)
    idxcat = jnp.concatenate(
        [idxcat, jnp.zeros((B - ntot,), dtype=idxcat.dtype)])
    g = _sc_gather(table, idxcat.astype(jnp.int32), B)
    n20 = idx20.shape[0]
    n40 = idx40d.shape[0]
    n9 = idx9.shape[0]
    return g[:n20], g[n20:n20 + n40], g[n20 + n40:n20 + n40 + n9]


def _edge_body(q1g, q2g, f9, p12, feat, w1d, b1d, w2d, b2d, lw, lin_b, out_o):
    er20 = lax.broadcasted_iota(jnp.int32, (R_EDGE * 20, R_EDGE), 0) // 20
    ec20 = lax.broadcasted_iota(jnp.int32, (R_EDGE * 20, R_EDGE), 1)
    e20 = (er20 == ec20).astype(jnp.float32)

    q12g = jnp.concatenate([q1g[...], q2g[...]], axis=1)
    pexp = jnp.dot(e20, p12[...], preferred_element_type=jnp.float32)
    h = _leaky(pexp - q12g)
    h = _leaky(jnp.dot(h, w1d[...], preferred_element_type=jnp.float32) + b1d[...])
    h = jnp.dot(h, w2d[...], preferred_element_type=jnp.float32) + b2d[...]
    h12 = jnp.max(h.reshape(R_EDGE, 20, 2 * D), axis=1)
    hm = jnp.max(f9[...].reshape(R_EDGE, 9, D), axis=1)
    hcat = jnp.concatenate([h12, hm], axis=1)
    out = (jnp.dot(hcat, lw[...], preferred_element_type=jnp.float32)
           + lin_b[...] + feat[...])
    out_o[...] = out


def _edge(q1g, q2g, f9, p12, feat, w1d, b1d, w2d, b2d, lw, lin_b):
    nb = N // R_EDGE
    full = lambda shape: pl.BlockSpec(shape, lambda i: tuple(0 for _ in shape))
    return pl.pallas_call(
        _edge_body,
        grid=(nb,),
        in_specs=[
            pl.BlockSpec((R_EDGE * 20, D), lambda i: (i, 0)),
            pl.BlockSpec((R_EDGE * 20, D), lambda i: (i, 0)),
            pl.BlockSpec((R_EDGE * 9, D), lambda i: (i, 0)),
            pl.BlockSpec((R_EDGE, 2 * D), lambda i: (i, 0)),
            pl.BlockSpec((R_EDGE, IN_FEATS), lambda i: (i, 0)),
            full((2 * D, 2 * HID)), full((1, 2 * HID)),
            full((2 * HID, 2 * D)), full((1, 2 * D)),
            full((3 * D, IN_FEATS)), full((1, IN_FEATS)),
        ],
        out_specs=pl.BlockSpec((R_EDGE, IN_FEATS), lambda i: (i, 0)),
        out_shape=jax.ShapeDtypeStruct((N, IN_FEATS), jnp.float32),
    )(q1g, q2g, f9, p12, feat, w1d, b1d, w2d, b2d, lw, lin_b)


def kernel(feature, btn_w, btn_b,
           g1_theta_w, g1_theta_b, g1_phi_w, g1_phi_b,
           g1_mlp_w1, g1_mlp_b1, g1_mlp_w2, g1_mlp_b2,
           g2_theta_w, g2_theta_b, g2_phi_w, g2_phi_b,
           g2_mlp_w1, g2_mlp_b1, g2_mlp_w2, g2_mlp_b2,
           lin_w, lin_b):
    r = lambda b: b.reshape(1, -1)
    tpw1 = g1_theta_w + g1_phi_w
    tpb1 = r(g1_theta_b + g1_phi_b)
    tpw2 = g2_theta_w + g2_phi_w
    tpb2 = r(g2_theta_b + g2_phi_b)

    f, sq, p1, q1, p2, q2 = _prep(feature, btn_w, r(btn_b),
                                  tpw1, tpb1, g1_theta_w,
                                  tpw2, tpb2, g2_theta_w)
    sqT = sq.reshape(1, N)
    idx = _knn(f, sq, sqT)

    idx20 = idx[:, :20].reshape(-1)
    idx40d = idx[:, ::2].reshape(-1)
    idx9 = idx[:, :9].reshape(-1)
    q1g, q2g, f9 = _gather_all(q1, q2, f, idx20, idx40d, idx9)

    p12 = jnp.concatenate([p1, p2], axis=1)
    z = jnp.zeros((D, HID), jnp.float32)
    w1d = jnp.block([[g1_mlp_w1, z], [z, g2_mlp_w1]])
    b1d = jnp.concatenate([g1_mlp_b1, g2_mlp_b1]).reshape(1, -1)
    z2 = jnp.zeros((HID, D), jnp.float32)
    w2d = jnp.block([[g1_mlp_w2, z2], [z2, g2_mlp_w2]])
    b2d = jnp.concatenate([g1_mlp_b2, g2_mlp_b2]).reshape(1, -1)
    out = _edge(q1g, q2g, f9, p12, feature,
                w1d, b1d, w2d, b2d, lin_w, r(lin_b))
    return out


# speedup vs baseline: 18.3460x; 1.0065x over previous
import functools
import jax
import jax.numpy as jnp
from jax import lax
from jax.experimental import pallas as pl
from jax.experimental.pallas import tpu as pltpu
from jax.experimental.pallas import tpu_sc as plsc

N = 10000
IN_FEATS = 256
D = 64
HID = 64
K = 40

R_PREP = 1000
R_KNN = 200
R_EDGE = 200


def _leaky(x):
    return jnp.where(x >= 0, x, 0.01 * x)


def _prep_body(feat, btn_w, btn_b, tpw1, tpb1, tw1, tpw2, tpb2, tw2,
               f_o, sq_o, p1_o, q1_o, p2_o, q2_o):
    f = jnp.dot(feat[...], btn_w[...], preferred_element_type=jnp.float32) + btn_b[...]
    f_o[...] = f
    sq_o[...] = jnp.sum(f * f, axis=1, keepdims=True)
    p1_o[...] = jnp.dot(f, tpw1[...], preferred_element_type=jnp.float32) + tpb1[...]
    q1_o[...] = jnp.dot(f, tw1[...], preferred_element_type=jnp.float32)
    p2_o[...] = jnp.dot(f, tpw2[...], preferred_element_type=jnp.float32) + tpb2[...]
    q2_o[...] = jnp.dot(f, tw2[...], preferred_element_type=jnp.float32)


def _prep(feature, btn_w, btn_b, tpw1, tpb1, tw1, tpw2, tpb2, tw2):
    nb = N // R_PREP
    full = lambda shape: pl.BlockSpec(shape, lambda i: tuple(0 for _ in shape))
    out_shapes = (
        jax.ShapeDtypeStruct((N, D), jnp.float32),
        jax.ShapeDtypeStruct((N, 1), jnp.float32),
        jax.ShapeDtypeStruct((N, D), jnp.float32),
        jax.ShapeDtypeStruct((N, D), jnp.float32),
        jax.ShapeDtypeStruct((N, D), jnp.float32),
        jax.ShapeDtypeStruct((N, D), jnp.float32),
    )
    blk_nd = pl.BlockSpec((R_PREP, D), lambda i: (i, 0))
    return pl.pallas_call(
        _prep_body,
        grid=(nb,),
        in_specs=[
            pl.BlockSpec((R_PREP, IN_FEATS), lambda i: (i, 0)),
            full((IN_FEATS, D)), full((1, D)),
            full((D, D)), full((1, D)), full((D, D)),
            full((D, D)), full((1, D)), full((D, D)),
        ],
        out_specs=(
            blk_nd, pl.BlockSpec((R_PREP, 1), lambda i: (i, 0)),
            blk_nd, blk_nd, blk_nd, blk_nd,
        ),
        out_shape=out_shapes,
    )(feature, btn_w, btn_b, tpw1, tpb1, tw1, tpw2, tpb2, tw2)


NLANE = 128
NCHUNK = (N + NLANE - 1) // NLANE
B_INS = 6


def _knn_body(f_blk, sq_blk, f_full, sqT, idx_o, dscr):
    R = R_KNN
    g = lax.dot_general(f_blk[...], f_full[...],
                        (((1,), (1,)), ((), ())),
                        preferred_element_type=jnp.float32)
    dist = (sq_blk[...] + sqT[...]) - 2.0 * g
    big = jnp.int32(2 ** 30)
    inf = jnp.float32(jnp.inf)

    pad = NCHUNK * NLANE - N
    dscr[...] = jnp.concatenate(
        [dist, jnp.full((R, pad), inf, dtype=jnp.float32)], axis=1)

    lane = lax.broadcasted_iota(jnp.int32, (R, NLANE), 1)

    def lex_lt(av, ai, bv, bi):
        return (av < bv) | ((av == bv) & (ai < bi))

    def ins_step(c, st):
        x = dscr[:, pl.ds(c * NLANE, NLANE)]
        xi = lane + c * NLANE
        cv, ci = x, xi
        inserted = None
        out = []
        for b in range(B_INS):
            vb, ib = st[2 * b], st[2 * b + 1]
            lt = cv < vb
            if inserted is not None:
                lt = inserted | lt
            nv = jnp.where(lt, cv, vb)
            ni = jnp.where(lt, ci, ib)
            cv, ci = jnp.where(lt, vb, cv), jnp.where(lt, ib, ci)
            inserted = lt
            out += [nv, ni]
        return tuple(out)

    init = []
    for b in range(B_INS):
        init += [jnp.full((R, NLANE), inf, dtype=jnp.float32),
                 jnp.full((R, NLANE), big, dtype=jnp.int32)]
    st = lax.fori_loop(0, NCHUNK, ins_step, tuple(init), unroll=16)

    vs = [st[2 * b] for b in range(B_INS)]
    is_ = [st[2 * b + 1] for b in range(B_INS)]
    cols = []
    m = None
    amin = None
    for t in range(K):
        m = jnp.min(vs[0], axis=1, keepdims=True)
        amin = jnp.min(jnp.where(vs[0] == m, is_[0], big), axis=1, keepdims=True)
        cols.append(amin)
        lm = is_[0] == amin
        for b in range(B_INS - 1):
            vs[b] = jnp.where(lm, vs[b + 1], vs[b])
            is_[b] = jnp.where(lm, is_[b + 1], is_[b])
        vs[B_INS - 1] = jnp.where(lm, inf, vs[B_INS - 1])
        is_[B_INS - 1] = jnp.where(lm, big, is_[B_INS - 1])
    idx_fast = jnp.concatenate(cols, axis=1)

    ovf = jnp.any(lex_lt(st[2 * (B_INS - 1)], st[2 * (B_INS - 1) + 1], m, amin))

    def slow():
        iota = lax.broadcasted_iota(jnp.int32, (R, N), 1)
        d = dist
        cs = []
        for t in range(K):
            mm = jnp.min(d, axis=1, keepdims=True)
            am = jnp.min(jnp.where(d == mm, iota, big), axis=1, keepdims=True)
            cs.append(am)
            d = jnp.where(iota == am, inf, d)
        return jnp.concatenate(cs, axis=1)

    idx_o[...] = lax.cond(ovf, slow, lambda: idx_fast)


def _knn(f, sq, sqT):
    nb = N // R_KNN
    return pl.pallas_call(
        _knn_body,
        grid=(nb,),
        in_specs=[
            pl.BlockSpec((R_KNN, D), lambda i: (i, 0)),
            pl.BlockSpec((R_KNN, 1), lambda i: (i, 0)),
            pl.BlockSpec((N, D), lambda i: (0, 0)),
            pl.BlockSpec((1, N), lambda i: (0, 0)),
        ],
        out_specs=pl.BlockSpec((R_KNN, K), lambda i: (i, 0)),
        out_shape=jax.ShapeDtypeStruct((N, K), jnp.int32),
        scratch_shapes=[pltpu.VMEM((R_KNN, NCHUNK * NLANE), jnp.float32)],
    )(f, sq, f, sqT)


NW = 32
GCH = 128


def _sc_gather(table, idx, B):
    bpw = B // NW
    nch = bpw // GCH
    mesh = plsc.VectorSubcoreMesh(core_axis_name="c", subcore_axis_name="s")

    @functools.partial(
        pl.kernel, mesh=mesh,
        compiler_params=pltpu.CompilerParams(use_tc_tiling_on_sc=False),
        out_type=jax.ShapeDtypeStruct((B, D), jnp.float32),
        scratch_types=[
            pltpu.VMEM((bpw,), jnp.int32),
            pltpu.VMEM((GCH, D), jnp.float32),
            pltpu.VMEM((GCH, D), jnp.float32),
            pltpu.SemaphoreType.DMA,
            pltpu.SemaphoreType.DMA,
        ],
    )
    def k(table_hbm, idx_hbm, out_hbm, idx_v, buf0, buf1, sem0, sem1):
        wid = lax.axis_index("s") * 2 + lax.axis_index("c")
        base = wid * bpw
        pltpu.sync_copy(idx_hbm.at[pl.ds(base, bpw)], idx_v)

        def start(g, buf, sem):
            pltpu.async_copy(table_hbm.at[idx_v.at[pl.ds(g * GCH, GCH)]], buf, sem)

        def wait(buf, sem):
            pltpu.make_async_copy(table_hbm.at[pl.ds(0, GCH)], buf, sem).wait()

        start(0, buf0, sem0)

        def body2(h, _):
            g0 = 2 * h
            start(g0 + 1, buf1, sem1)
            wait(buf0, sem0)
            pltpu.sync_copy(buf0, out_hbm.at[pl.ds(base + g0 * GCH, GCH)])

            @pl.when(g0 + 2 < nch)
            def _():
                start(g0 + 2, buf0, sem0)

            wait(buf1, sem1)
            pltpu.sync_copy(buf1, out_hbm.at[pl.ds(base + (g0 + 1) * GCH, GCH)])
            return 0

        lax.fori_loop(0, nch // 2, body2, 0)

    return k(table, idx)


def _gather_all(q1, q2, f, idx20, idx40d, idx9):
    table = jnp.concatenate([q1, q2, f], axis=0)
    idxcat = jnp.concatenate([idx20, idx40d + N, idx9 + 2 * N])
    ntot = idxcat.shape[0]
    B = ((ntot + NW * GCH - 1) // (NW * GCH)) * (NW * GCH)
    idxcat = jnp.concatenate(
        [idxcat, jnp.zeros((B - ntot,), dtype=idxcat.dtype)])
    g = _sc_gather(table, idxcat.astype(jnp.int32), B)
    n20 = idx20.shape[0]
    n40 = idx40d.shape[0]
    n9 = idx9.shape[0]
    return g[:n20], g[n20:n20 + n40], g[n20 + n40:n20 + n40 + n9]


def _edge_body(q1g, q2g, f9, p12, feat, w1d, b1d, w2d, b2d, lw, lin_b, out_o):
    q12g = jnp.concatenate([q1g[...], q2g[...]], axis=1)
    pexp = jnp.broadcast_to(
        p12[...][:, None, :], (R_EDGE, 20, 2 * D)).reshape(R_EDGE * 20, 2 * D)
    h = _leaky(pexp - q12g)
    h = _leaky(jnp.dot(h, w1d[...], preferred_element_type=jnp.float32) + b1d[...])
    h = jnp.dot(h, w2d[...], preferred_element_type=jnp.float32) + b2d[...]
    h12 = jnp.max(h.reshape(R_EDGE, 20, 2 * D), axis=1)
    hm = jnp.max(f9[...].reshape(R_EDGE, 9, D), axis=1)
    hcat = jnp.concatenate([h12, hm], axis=1)
    out = (jnp.dot(hcat, lw[...], preferred_element_type=jnp.float32)
           + lin_b[...] + feat[...])
    out_o[...] = out


def _edge(q1g, q2g, f9, p12, feat, w1d, b1d, w2d, b2d, lw, lin_b):
    nb = N // R_EDGE
    full = lambda shape: pl.BlockSpec(shape, lambda i: tuple(0 for _ in shape))
    return pl.pallas_call(
        _edge_body,
        grid=(nb,),
        in_specs=[
            pl.BlockSpec((R_EDGE * 20, D), lambda i: (i, 0)),
            pl.BlockSpec((R_EDGE * 20, D), lambda i: (i, 0)),
            pl.BlockSpec((R_EDGE * 9, D), lambda i: (i, 0)),
            pl.BlockSpec((R_EDGE, 2 * D), lambda i: (i, 0)),
            pl.BlockSpec((R_EDGE, IN_FEATS), lambda i: (i, 0)),
            full((2 * D, 2 * HID)), full((1, 2 * HID)),
            full((2 * HID, 2 * D)), full((1, 2 * D)),
            full((3 * D, IN_FEATS)), full((1, IN_FEATS)),
        ],
        out_specs=pl.BlockSpec((R_EDGE, IN_FEATS), lambda i: (i, 0)),
        out_shape=jax.ShapeDtypeStruct((N, IN_FEATS), jnp.float32),
    )(q1g, q2g, f9, p12, feat, w1d, b1d, w2d, b2d, lw, lin_b)


def kernel(feature, btn_w, btn_b,
           g1_theta_w, g1_theta_b, g1_phi_w, g1_phi_b,
           g1_mlp_w1, g1_mlp_b1, g1_mlp_w2, g1_mlp_b2,
           g2_theta_w, g2_theta_b, g2_phi_w, g2_phi_b,
           g2_mlp_w1, g2_mlp_b1, g2_mlp_w2, g2_mlp_b2,
           lin_w, lin_b):
    r = lambda b: b.reshape(1, -1)
    tpw1 = g1_theta_w + g1_phi_w
    tpb1 = r(g1_theta_b + g1_phi_b)
    tpw2 = g2_theta_w + g2_phi_w
    tpb2 = r(g2_theta_b + g2_phi_b)

    f, sq, p1, q1, p2, q2 = _prep(feature, btn_w, r(btn_b),
                                  tpw1, tpb1, g1_theta_w,
                                  tpw2, tpb2, g2_theta_w)
    sqT = sq.reshape(1, N)
    idx = _knn(f, sq, sqT)

    idx20 = idx[:, :20].reshape(-1)
    idx40d = idx[:, ::2].reshape(-1)
    idx9 = idx[:, :9].reshape(-1)
    q1g, q2g, f9 = _gather_all(q1, q2, f, idx20, idx40d, idx9)

    p12 = jnp.concatenate([p1, p2], axis=1)
    z = jnp.zeros((D, HID), jnp.float32)
    w1d = jnp.block([[g1_mlp_w1, z], [z, g2_mlp_w1]])
    b1d = jnp.concatenate([g1_mlp_b1, g2_mlp_b1]).reshape(1, -1)
    z2 = jnp.zeros((HID, D), jnp.float32)
    w2d = jnp.block([[g1_mlp_w2, z2], [z2, g2_mlp_w2]])
    b2d = jnp.concatenate([g1_mlp_b2, g2_mlp_b2]).reshape(1, -1)
    out = _edge(q1g, q2g, f9, p12, feature,
                w1d, b1d, w2d, b2d, lin_w, r(lin_b))
    return out


# speedup vs baseline: 18.3939x; 1.0026x over previous
import functools
import jax
import jax.numpy as jnp
from jax import lax
from jax.experimental import pallas as pl
from jax.experimental.pallas import tpu as pltpu
from jax.experimental.pallas import tpu_sc as plsc

N = 10000
IN_FEATS = 256
D = 64
HID = 64
K = 40

R_PREP = 1000
R_KNN = 200
R_EDGE = 200


def _leaky(x):
    return jnp.maximum(x, 0.01 * x)


def _prep_body(feat, btn_w, btn_b, tpw1, tpb1, tw1, tpw2, tpb2, tw2,
               f_o, sq_o, p1_o, q1_o, p2_o, q2_o):
    f = jnp.dot(feat[...], btn_w[...], preferred_element_type=jnp.float32) + btn_b[...]
    f_o[...] = f
    sq_o[...] = jnp.sum(f * f, axis=1, keepdims=True)
    p1_o[...] = jnp.dot(f, tpw1[...], preferred_element_type=jnp.float32) + tpb1[...]
    q1_o[...] = jnp.dot(f, tw1[...], preferred_element_type=jnp.float32)
    p2_o[...] = jnp.dot(f, tpw2[...], preferred_element_type=jnp.float32) + tpb2[...]
    q2_o[...] = jnp.dot(f, tw2[...], preferred_element_type=jnp.float32)


def _prep(feature, btn_w, btn_b, tpw1, tpb1, tw1, tpw2, tpb2, tw2):
    nb = N // R_PREP
    full = lambda shape: pl.BlockSpec(shape, lambda i: tuple(0 for _ in shape))
    out_shapes = (
        jax.ShapeDtypeStruct((N, D), jnp.float32),
        jax.ShapeDtypeStruct((N, 1), jnp.float32),
        jax.ShapeDtypeStruct((N, D), jnp.float32),
        jax.ShapeDtypeStruct((N, D), jnp.float32),
        jax.ShapeDtypeStruct((N, D), jnp.float32),
        jax.ShapeDtypeStruct((N, D), jnp.float32),
    )
    blk_nd = pl.BlockSpec((R_PREP, D), lambda i: (i, 0))
    return pl.pallas_call(
        _prep_body,
        grid=(nb,),
        in_specs=[
            pl.BlockSpec((R_PREP, IN_FEATS), lambda i: (i, 0)),
            full((IN_FEATS, D)), full((1, D)),
            full((D, D)), full((1, D)), full((D, D)),
            full((D, D)), full((1, D)), full((D, D)),
        ],
        out_specs=(
            blk_nd, pl.BlockSpec((R_PREP, 1), lambda i: (i, 0)),
            blk_nd, blk_nd, blk_nd, blk_nd,
        ),
        out_shape=out_shapes,
    )(feature, btn_w, btn_b, tpw1, tpb1, tw1, tpw2, tpb2, tw2)


NLANE = 128
NCHUNK = (N + NLANE - 1) // NLANE
B_INS = 6


def _knn_body(f_blk, sq_blk, f_full, sqT, idx_o, dscr):
    R = R_KNN
    g = lax.dot_general(f_blk[...], f_full[...],
                        (((1,), (1,)), ((), ())),
                        preferred_element_type=jnp.float32)
    dist = (sq_blk[...] + sqT[...]) - 2.0 * g
    big = jnp.int32(2 ** 30)
    inf = jnp.float32(jnp.inf)

    pad = NCHUNK * NLANE - N
    dscr[...] = jnp.concatenate(
        [dist, jnp.full((R, pad), inf, dtype=jnp.float32)], axis=1)

    lane = lax.broadcasted_iota(jnp.int32, (R, NLANE), 1)

    def lex_lt(av, ai, bv, bi):
        return (av < bv) | ((av == bv) & (ai < bi))

    def ins_step(c, st):
        x = dscr[:, pl.ds(c * NLANE, NLANE)]
        xi = lane + c * NLANE
        cv, ci = x, xi
        inserted = None
        out = []
        for b in range(B_INS):
            vb, ib = st[2 * b], st[2 * b + 1]
            lt = cv < vb
            if inserted is not None:
                lt = inserted | lt
            nv = jnp.where(lt, cv, vb)
            ni = jnp.where(lt, ci, ib)
            cv, ci = jnp.where(lt, vb, cv), jnp.where(lt, ib, ci)
            inserted = lt
            out += [nv, ni]
        return tuple(out)

    init = []
    for b in range(B_INS):
        init += [jnp.full((R, NLANE), inf, dtype=jnp.float32),
                 jnp.full((R, NLANE), big, dtype=jnp.int32)]
    st = lax.fori_loop(0, NCHUNK, ins_step, tuple(init), unroll=16)

    vs = [st[2 * b] for b in range(B_INS)]
    is_ = [st[2 * b + 1] for b in range(B_INS)]
    cols = []
    m = None
    amin = None
    for t in range(K):
        m = jnp.min(vs[0], axis=1, keepdims=True)
        amin = jnp.min(jnp.where(vs[0] == m, is_[0], big), axis=1, keepdims=True)
        cols.append(amin)
        lm = is_[0] == amin
        for b in range(B_INS - 1):
            vs[b] = jnp.where(lm, vs[b + 1], vs[b])
            is_[b] = jnp.where(lm, is_[b + 1], is_[b])
        vs[B_INS - 1] = jnp.where(lm, inf, vs[B_INS - 1])
        is_[B_INS - 1] = jnp.where(lm, big, is_[B_INS - 1])
    idx_fast = jnp.concatenate(cols, axis=1)

    ovf = jnp.any(lex_lt(st[2 * (B_INS - 1)], st[2 * (B_INS - 1) + 1], m, amin))

    def slow():
        iota = lax.broadcasted_iota(jnp.int32, (R, N), 1)
        d = dist
        cs = []
        for t in range(K):
            mm = jnp.min(d, axis=1, keepdims=True)
            am = jnp.min(jnp.where(d == mm, iota, big), axis=1, keepdims=True)
            cs.append(am)
            d = jnp.where(iota == am, inf, d)
        return jnp.concatenate(cs, axis=1)

    idx_o[...] = lax.cond(ovf, slow, lambda: idx_fast)


def _knn(f, sq, sqT):
    nb = N // R_KNN
    return pl.pallas_call(
        _knn_body,
        grid=(nb,),
        in_specs=[
            pl.BlockSpec((R_KNN, D), lambda i: (i, 0)),
            pl.BlockSpec((R_KNN, 1), lambda i: (i, 0)),
            pl.BlockSpec((N, D), lambda i: (0, 0)),
            pl.BlockSpec((1, N), lambda i: (0, 0)),
        ],
        out_specs=pl.BlockSpec((R_KNN, K), lambda i: (i, 0)),
        out_shape=jax.ShapeDtypeStruct((N, K), jnp.int32),
        scratch_shapes=[pltpu.VMEM((R_KNN, NCHUNK * NLANE), jnp.float32)],
    )(f, sq, f, sqT)


NW = 32
GCH = 128


def _sc_gather(table, idx, B):
    bpw = B // NW
    nch = bpw // GCH
    mesh = plsc.VectorSubcoreMesh(core_axis_name="c", subcore_axis_name="s")

    @functools.partial(
        pl.kernel, mesh=mesh,
        compiler_params=pltpu.CompilerParams(use_tc_tiling_on_sc=False),
        out_type=jax.ShapeDtypeStruct((B, D), jnp.float32),
        scratch_types=[
            pltpu.VMEM((bpw,), jnp.int32),
            pltpu.VMEM((GCH, D), jnp.float32),
            pltpu.VMEM((GCH, D), jnp.float32),
            pltpu.SemaphoreType.DMA,
            pltpu.SemaphoreType.DMA,
        ],
    )
    def k(table_hbm, idx_hbm, out_hbm, idx_v, buf0, buf1, sem0, sem1):
        wid = lax.axis_index("s") * 2 + lax.axis_index("c")
        base = wid * bpw
        pltpu.sync_copy(idx_hbm.at[pl.ds(base, bpw)], idx_v)

        def start(g, buf, sem):
            pltpu.async_copy(table_hbm.at[idx_v.at[pl.ds(g * GCH, GCH)]], buf, sem)

        def wait(buf, sem):
            pltpu.make_async_copy(table_hbm.at[pl.ds(0, GCH)], buf, sem).wait()

        start(0, buf0, sem0)

        def body2(h, _):
            g0 = 2 * h
            start(g0 + 1, buf1, sem1)
            wait(buf0, sem0)
            pltpu.sync_copy(buf0, out_hbm.at[pl.ds(base + g0 * GCH, GCH)])

            @pl.when(g0 + 2 < nch)
            def _():
                start(g0 + 2, buf0, sem0)

            wait(buf1, sem1)
            pltpu.sync_copy(buf1, out_hbm.at[pl.ds(base + (g0 + 1) * GCH, GCH)])
            return 0

        lax.fori_loop(0, nch // 2, body2, 0)

    return k(table, idx)


def _gather_all(q1, q2, f, idx20, idx40d, idx9):
    table = jnp.concatenate([q1, q2, f], axis=0)
    idxcat = jnp.concatenate([idx20, idx40d + N, idx9 + 2 * N])
    ntot = idxcat.shape[0]
    B = ((ntot + NW * GCH - 1) // (NW * GCH)) * (NW * GCH)
    idxcat = jnp.concatenate(
        [idxcat, jnp.zeros((B - ntot,), dtype=idxcat.dtype)])
    g = _sc_gather(table, idxcat.astype(jnp.int32), B)
    n20 = idx20.shape[0]
    n40 = idx40d.shape[0]
    n9 = idx9.shape[0]
    return g[:n20], g[n20:n20 + n40], g[n20 + n40:n20 + n40 + n9]


def _edge_body(q1g, q2g, f9, p12, feat, w1d, b1d, w2d, b2d, lw, lin_b, out_o):
    q12g = jnp.concatenate([q1g[...], q2g[...]], axis=1)
    pexp = jnp.broadcast_to(
        p12[...][:, None, :], (R_EDGE, 20, 2 * D)).reshape(R_EDGE * 20, 2 * D)
    h = _leaky(pexp - q12g)
    h = _leaky(jnp.dot(h, w1d[...], preferred_element_type=jnp.float32) + b1d[...])
    h = jnp.dot(h, w2d[...], preferred_element_type=jnp.float32) + b2d[...]
    h12 = jnp.max(h.reshape(R_EDGE, 20, 2 * D), axis=1)
    hm = jnp.max(f9[...].reshape(R_EDGE, 9, D), axis=1)
    hcat = jnp.concatenate([h12, hm], axis=1)
    out = (jnp.dot(hcat, lw[...], preferred_element_type=jnp.float32)
           + lin_b[...] + feat[...])
    out_o[...] = out


def _edge(q1g, q2g, f9, p12, feat, w1d, b1d, w2d, b2d, lw, lin_b):
    nb = N // R_EDGE
    full = lambda shape: pl.BlockSpec(shape, lambda i: tuple(0 for _ in shape))
    return pl.pallas_call(
        _edge_body,
        grid=(nb,),
        in_specs=[
            pl.BlockSpec((R_EDGE * 20, D), lambda i: (i, 0)),
            pl.BlockSpec((R_EDGE * 20, D), lambda i: (i, 0)),
            pl.BlockSpec((R_EDGE * 9, D), lambda i: (i, 0)),
            pl.BlockSpec((R_EDGE, 2 * D), lambda i: (i, 0)),
            pl.BlockSpec((R_EDGE, IN_FEATS), lambda i: (i, 0)),
            full((2 * D, 2 * HID)), full((1, 2 * HID)),
            full((2 * HID, 2 * D)), full((1, 2 * D)),
            full((3 * D, IN_FEATS)), full((1, IN_FEATS)),
        ],
        out_specs=pl.BlockSpec((R_EDGE, IN_FEATS), lambda i: (i, 0)),
        out_shape=jax.ShapeDtypeStruct((N, IN_FEATS), jnp.float32),
    )(q1g, q2g, f9, p12, feat, w1d, b1d, w2d, b2d, lw, lin_b)


def kernel(feature, btn_w, btn_b,
           g1_theta_w, g1_theta_b, g1_phi_w, g1_phi_b,
           g1_mlp_w1, g1_mlp_b1, g1_mlp_w2, g1_mlp_b2,
           g2_theta_w, g2_theta_b, g2_phi_w, g2_phi_b,
           g2_mlp_w1, g2_mlp_b1, g2_mlp_w2, g2_mlp_b2,
           lin_w, lin_b):
    r = lambda b: b.reshape(1, -1)
    tpw1 = g1_theta_w + g1_phi_w
    tpb1 = r(g1_theta_b + g1_phi_b)
    tpw2 = g2_theta_w + g2_phi_w
    tpb2 = r(g2_theta_b + g2_phi_b)

    f, sq, p1, q1, p2, q2 = _prep(feature, btn_w, r(btn_b),
                                  tpw1, tpb1, g1_theta_w,
                                  tpw2, tpb2, g2_theta_w)
    sqT = sq.reshape(1, N)
    idx = _knn(f, sq, sqT)

    idx20 = idx[:, :20].reshape(-1)
    idx40d = idx[:, ::2].reshape(-1)
    idx9 = idx[:, :9].reshape(-1)
    q1g, q2g, f9 = _gather_all(q1, q2, f, idx20, idx40d, idx9)

    p12 = jnp.concatenate([p1, p2], axis=1)
    z = jnp.zeros((D, HID), jnp.float32)
    w1d = jnp.block([[g1_mlp_w1, z], [z, g2_mlp_w1]])
    b1d = jnp.concatenate([g1_mlp_b1, g2_mlp_b1]).reshape(1, -1)
    z2 = jnp.zeros((HID, D), jnp.float32)
    w2d = jnp.block([[g1_mlp_w2, z2], [z2, g2_mlp_w2]])
    b2d = jnp.concatenate([g1_mlp_b2, g2_mlp_b2]).reshape(1, -1)
    out = _edge(q1g, q2g, f9, p12, feature,
                w1d, b1d, w2d, b2d, lin_w, r(lin_b))
    return out
